# Initial kernel scaffold; baseline (speedup 1.0000x reference)
import jax
import jax.numpy as jnp
from jax.experimental import pallas as pl


def kernel(x, W1_self, W1_neigh, b1, W2_self, W2_neigh, b2, W_lin1, b_lin1, W_lin2, b_lin2, n_id, edge_index):
    raise NotImplementedError("write your pallas kernel here")


# speedup vs baseline: 5.9191x; 5.9191x over previous
import functools

import jax
import jax.numpy as jnp
from jax import lax
from jax.experimental import pallas as pl
from jax.experimental.pallas import tpu as pltpu
from jax.experimental.pallas import tpu_sc as plsc

N = 10000
E = 320000
D = 128
H = 128
C = 64

NC = 2
NS = 16
NW = NC * NS
EPT = E // NW
EB = 128
NFULL = EPT // EB
ETAIL = EPT - NFULL * EB
RPS = 624
RTAIL = N - NS * RPS
GB = 128
NGB = N // GB
GTAIL = N - NGB * GB

_MESH = plsc.VectorSubcoreMesh(
    core_axis_name="c", subcore_axis_name="s", num_cores=NC, num_subcores=NS)


def _fill_ones(ref, n):
    for i in range(n // 16):
        ref[pl.ds(i * 16, 16)] = jnp.full((16,), 1.0, jnp.float32)


def _edge_batch(table, n_id, src_h, dst_h, base, nb, idx_v, gidx_v, dst_v,
                rows_v, ones_v, sem, agg_s, deg_s, compose):
    pltpu.sync_copy(src_h.at[pl.ds(base, nb)], idx_v)
    if compose:
        pltpu.async_copy(n_id.at[idx_v], gidx_v, sem).wait()
        row_idx = gidx_v
    else:
        row_idx = idx_v
    pltpu.async_copy(table.at[row_idx], rows_v, sem).wait()
    pltpu.sync_copy(dst_h.at[pl.ds(base, nb)], dst_v)
    pltpu.sync_copy(rows_v, agg_s.at[dst_v], add=True)
    if deg_s is not None:
        pltpu.sync_copy(ones_v, deg_s.at[dst_v], add=True)


def _sage_agg1_kernel():
    @functools.partial(
        pl.kernel,
        out_type=(
            jax.ShapeDtypeStruct((N, D), jnp.float32),
            jax.ShapeDtypeStruct((NC * N,), jnp.float32),
            jax.ShapeDtypeStruct((NC, N, D), jnp.float32),
        ),
        mesh=_MESH,
        scratch_types=[
            pltpu.VMEM((EB,), jnp.int32),
            pltpu.VMEM((EB,), jnp.int32),
            pltpu.VMEM((EB,), jnp.int32),
            pltpu.VMEM((EB, D), jnp.float32),
            pltpu.VMEM((EB,), jnp.float32),
            pltpu.VMEM((ETAIL,), jnp.int32),
            pltpu.VMEM((ETAIL,), jnp.int32),
            pltpu.VMEM((ETAIL,), jnp.int32),
            pltpu.VMEM((ETAIL, D), jnp.float32),
            pltpu.VMEM((ETAIL,), jnp.float32),
            pltpu.VMEM((N,), jnp.float32),
            pltpu.VMEM_SHARED((N, D), jnp.float32),
            pltpu.VMEM_SHARED((N,), jnp.float32),
            pltpu.SemaphoreType.DMA,
        ],
    )
    def k(x_h, n_id_h, src_h, dst_h, zrows_h, zdeg_h,
          h0_h, deg_h, agg_h,
          idx_v, gidx_v, dst_v, rows_v, ones_v,
          idx_t, gidx_t, dst_t, rows_t, ones_t,
          deg_v, agg_s, deg_s, sem):
        c = lax.axis_index("c")
        s = lax.axis_index("s")
        wid = c * NS + s

        pltpu.sync_copy(zrows_h.at[pl.ds(s * RPS, RPS)], agg_s.at[pl.ds(s * RPS, RPS)])

        @pl.when(s == NS - 1)
        def _():
            pltpu.sync_copy(zrows_h.at[pl.ds(NS * RPS, RTAIL)],
                            agg_s.at[pl.ds(NS * RPS, RTAIL)])

        @pl.when(s == 0)
        def _():
            pltpu.sync_copy(zdeg_h, deg_v)
            pltpu.sync_copy(deg_v, deg_s)

        _fill_ones(ones_v, EB)
        _fill_ones(ones_t, ETAIL)
        plsc.subcore_barrier()

        def body(i, carry):
            _edge_batch(x_h, n_id_h, src_h, dst_h, wid * EPT + i * EB, EB,
                        idx_v, gidx_v, dst_v, rows_v, ones_v, sem,
                        agg_s, deg_s, compose=True)
            return carry

        lax.fori_loop(0, NFULL, body, 0)
        _edge_batch(x_h, n_id_h, src_h, dst_h, wid * EPT + NFULL * EB, ETAIL,
                    idx_t, gidx_t, dst_t, rows_t, ones_t, sem,
                    agg_s, deg_s, compose=True)

        for kk in range((NGB + NW - 1) // NW):
            b = wid + NW * kk

            @pl.when(b < NGB)
            def _():
                pltpu.sync_copy(n_id_h.at[pl.ds(b * GB, GB)], idx_v)
                pltpu.async_copy(x_h.at[idx_v], rows_v, sem).wait()
                pltpu.sync_copy(rows_v, h0_h.at[pl.ds(b * GB, GB)])

        @pl.when(wid == 0)
        def _():
            pltpu.sync_copy(n_id_h.at[pl.ds(NGB * GB, GTAIL)], idx_t)
            pltpu.async_copy(x_h.at[idx_t], rows_t, sem).wait()
            pltpu.sync_copy(rows_t, h0_h.at[pl.ds(NGB * GB, GTAIL)])

        plsc.subcore_barrier()

        pltpu.sync_copy(agg_s.at[pl.ds(s * RPS, RPS)], agg_h.at[c, pl.ds(s * RPS, RPS)])

        @pl.when(s == NS - 1)
        def _():
            pltpu.sync_copy(agg_s.at[pl.ds(NS * RPS, RTAIL)],
                            agg_h.at[c, pl.ds(NS * RPS, RTAIL)])

        @pl.when(s == 0)
        def _():
            pltpu.sync_copy(deg_s, deg_v)
            pltpu.sync_copy(deg_v, deg_h.at[pl.ds(c * N, N)])

    return k


def _sage_agg2_kernel():
    @functools.partial(
        pl.kernel,
        out_type=jax.ShapeDtypeStruct((NC, N, D), jnp.float32),
        mesh=_MESH,
        scratch_types=[
            pltpu.VMEM((EB,), jnp.int32),
            pltpu.VMEM((EB,), jnp.int32),
            pltpu.VMEM((EB, D), jnp.float32),
            pltpu.VMEM((ETAIL,), jnp.int32),
            pltpu.VMEM((ETAIL,), jnp.int32),
            pltpu.VMEM((ETAIL, D), jnp.float32),
            pltpu.VMEM_SHARED((N, D), jnp.float32),
            pltpu.SemaphoreType.DMA,
        ],
    )
    def k(h1_h, src_h, dst_h, zrows_h, agg_h,
          idx_v, dst_v, rows_v, idx_t, dst_t, rows_t, agg_s, sem):
        c = lax.axis_index("c")
        s = lax.axis_index("s")
        wid = c * NS + s

        pltpu.sync_copy(zrows_h.at[pl.ds(s * RPS, RPS)], agg_s.at[pl.ds(s * RPS, RPS)])

        @pl.when(s == NS - 1)
        def _():
            pltpu.sync_copy(zrows_h.at[pl.ds(NS * RPS, RTAIL)],
                            agg_s.at[pl.ds(NS * RPS, RTAIL)])

        plsc.subcore_barrier()

        def body(i, carry):
            _edge_batch(h1_h, None, src_h, dst_h, wid * EPT + i * EB, EB,
                        idx_v, None, dst_v, rows_v, None, sem,
                        agg_s, None, compose=False)
            return carry

        lax.fori_loop(0, NFULL, body, 0)
        _edge_batch(h1_h, None, src_h, dst_h, wid * EPT + NFULL * EB, ETAIL,
                    idx_t, None, dst_t, rows_t, None, sem,
                    agg_s, None, compose=False)

        plsc.subcore_barrier()
        pltpu.sync_copy(agg_s.at[pl.ds(s * RPS, RPS)], agg_h.at[c, pl.ds(s * RPS, RPS)])

        @pl.when(s == NS - 1)
        def _():
            pltpu.sync_copy(agg_s.at[pl.ds(NS * RPS, RTAIL)],
                            agg_h.at[c, pl.ds(NS * RPS, RTAIL)])

    return k


_R = 1000


def _tc1_body(h0, aggp, degp, ws, wn, b, out):
    agg = aggp[0] + aggp[1]
    deg = degp[0] + degp[1]
    mean = agg / jnp.maximum(deg, 1.0)
    acc = jnp.dot(h0[...], ws[...], preferred_element_type=jnp.float32)
    acc += jnp.dot(mean, wn[...], preferred_element_type=jnp.float32)
    out[...] = jnp.maximum(acc + b[...], 0.0)


def _tc2_body(h1, aggp, degp, w2s, w2n, b2, wl1, bl1, wl2, bl2, out):
    agg = aggp[0] + aggp[1]
    deg = degp[0] + degp[1]
    mean = agg / jnp.maximum(deg, 1.0)
    h2 = jnp.dot(h1[...], w2s[...], preferred_element_type=jnp.float32)
    h2 += jnp.dot(mean, w2n[...], preferred_element_type=jnp.float32)
    h2 = jnp.maximum(h2 + b2[...], 0.0)
    h3 = jnp.maximum(
        jnp.dot(h2, wl1[...], preferred_element_type=jnp.float32) + bl1[...], 0.0)
    logits = jnp.dot(h3, wl2[...], preferred_element_type=jnp.float32) + bl2[...]
    m = jnp.max(logits, axis=-1, keepdims=True)
    lse = jnp.log(jnp.sum(jnp.exp(logits - m), axis=-1, keepdims=True)) + m
    out[...] = logits - lse


def _row_specs():
    rows = pl.BlockSpec((_R, D), lambda i: (i, 0))
    aggp = pl.BlockSpec((NC, _R, D), lambda i: (0, i, 0))
    degp = pl.BlockSpec((NC, _R, 1), lambda i: (0, i, 0))
    w = pl.BlockSpec((D, D), lambda i: (0, 0))
    bias = pl.BlockSpec((1, D), lambda i: (0, 0))
    return rows, aggp, degp, w, bias


def _tc1_call(h0, agg, deg, ws, wn, b):
    rows, aggp, degp, w, bias = _row_specs()
    return pl.pallas_call(
        _tc1_body,
        grid=(N // _R,),
        in_specs=[rows, aggp, degp, w, w, bias],
        out_specs=rows,
        out_shape=jax.ShapeDtypeStruct((N, H), jnp.float32),
    )(h0, agg, deg.reshape(NC, N, 1), ws, wn, b.reshape(1, H))


def _tc2_call(h1, agg, deg, w2s, w2n, b2, wl1, bl1, wl2, bl2):
    rows, aggp, degp, w, bias = _row_specs()
    wc = pl.BlockSpec((H, C), lambda i: (0, 0))
    bc = pl.BlockSpec((1, C), lambda i: (0, 0))
    outc = pl.BlockSpec((_R, C), lambda i: (i, 0))
    return pl.pallas_call(
        _tc2_body,
        grid=(N // _R,),
        in_specs=[rows, aggp, degp, w, w, bias, w, bias, wc, bc],
        out_specs=outc,
        out_shape=jax.ShapeDtypeStruct((N, C), jnp.float32),
    )(h1, agg, deg.reshape(NC, N, 1), w2s, w2n, b2.reshape(1, H),
      wl1, bl1.reshape(1, H), wl2, bl2.reshape(1, C))


def kernel(x, W1_self, W1_neigh, b1, W2_self, W2_neigh, b2,
           W_lin1, b_lin1, W_lin2, b_lin2, n_id, edge_index):
    src = edge_index[0]
    dst = edge_index[1]
    zrows = jnp.zeros((N, D), jnp.float32)
    zdeg = jnp.zeros((N,), jnp.float32)

    h0, deg, agg1 = _sage_agg1_kernel()(x, n_id, src, dst, zrows, zdeg)
    deg = deg.reshape(NC, N)
    h1 = _tc1_call(h0, agg1, deg, W1_self, W1_neigh, b1)
    agg2 = _sage_agg2_kernel()(h1, src, dst, zrows)
    return _tc2_call(h1, agg2, deg, W2_self, W2_neigh, b2,
                     W_lin1, b_lin1, W_lin2, b_lin2)


# speedup vs baseline: 9.3645x; 1.5821x over previous
import functools

import jax
import jax.numpy as jnp
from jax import lax
from jax.experimental import pallas as pl
from jax.experimental.pallas import tpu as pltpu
from jax.experimental.pallas import tpu_sc as plsc

N = 10000
E = 320000
D = 128
H = 128
C = 64

NC = 2
NS = 16
NW = NC * NS
EPT = E // NW
EB = 128
NFULL = EPT // EB
ETAIL = EPT - NFULL * EB
RPS = 624
RTAIL = N - NS * RPS
GB = 128
NGB = N // GB
GTAIL = N - NGB * GB

_MESH = plsc.VectorSubcoreMesh(
    core_axis_name="c", subcore_axis_name="s", num_cores=NC, num_subcores=NS)


def _fill_ones(ref, n):
    for i in range(n // 16):
        ref[pl.ds(i * 16, 16)] = jnp.full((16,), 1.0, jnp.float32)


def _edge_pipeline(table, n_id_h, src_h, dst_h, wid,
                   idx_v, gidx_v, dst_v, rows_v, ones_v,
                   s_idx, s_dst, s_rows, s_gid, s_sca, s_dgs,
                   agg_s, deg_s, compose):
    def src_slice(i):
        return src_h.at[pl.ds(wid * EPT + i * EB, EB)]

    def dst_slice(i):
        return dst_h.at[pl.ds(wid * EPT + i * EB, EB)]

    def fire_loads(i, b):
        pltpu.async_copy(src_slice(i), idx_v[b], s_idx[b])
        pltpu.async_copy(dst_slice(i), dst_v[b], s_dst[b])

    fire_loads(0, 0)
    fire_loads(1, 1)

    def half_body(g, b):
        i = 2 * g + b
        b2 = 1 - b
        pltpu.make_async_copy(src_slice(i), idx_v[b], s_idx[b]).wait()
        if compose:
            pltpu.async_copy(n_id_h.at[idx_v[b]], gidx_v[b], s_gid[b]).wait()
            ridx = gidx_v[b]
        else:
            ridx = idx_v[b]
        pltpu.async_copy(table.at[ridx], rows_v[b], s_rows[b])

        def refill():
            pltpu.make_async_copy(rows_v[b2], agg_s.at[dst_v[b2]], s_sca[b2]).wait()
            if deg_s is not None:
                pltpu.make_async_copy(ones_v, deg_s.at[dst_v[b2]], s_dgs[b2]).wait()
            fire_loads(i + 1, b2)

        if b == 0:
            pl.when(g >= 1)(refill)
        else:
            pl.when(g < NFULL // 2 - 1)(refill)

        pltpu.make_async_copy(table.at[ridx], rows_v[b], s_rows[b]).wait()
        pltpu.make_async_copy(dst_slice(i), dst_v[b], s_dst[b]).wait()
        pltpu.async_copy(rows_v[b], agg_s.at[dst_v[b]], s_sca[b], add=True)
        if deg_s is not None:
            pltpu.async_copy(ones_v, deg_s.at[dst_v[b]], s_dgs[b], add=True)

    def body(g, carry):
        half_body(g, 0)
        half_body(g, 1)
        return carry

    lax.fori_loop(0, NFULL // 2, body, 0)

    for b in (0, 1):
        pltpu.make_async_copy(rows_v[b], agg_s.at[dst_v[b]], s_sca[b]).wait()
        if deg_s is not None:
            pltpu.make_async_copy(ones_v, deg_s.at[dst_v[b]], s_dgs[b]).wait()


def _edge_tail(table, n_id_h, src_h, dst_h, base,
               idx_t, gidx_t, dst_t, rows_t, ones_t, sem,
               agg_s, deg_s, compose):
    pltpu.sync_copy(src_h.at[pl.ds(base, ETAIL)], idx_t)
    if compose:
        pltpu.async_copy(n_id_h.at[idx_t], gidx_t, sem).wait()
        ridx = gidx_t
    else:
        ridx = idx_t
    pltpu.async_copy(table.at[ridx], rows_t, sem).wait()
    pltpu.sync_copy(dst_h.at[pl.ds(base, ETAIL)], dst_t)
    pltpu.sync_copy(rows_t, agg_s.at[dst_t], add=True)
    if deg_s is not None:
        pltpu.sync_copy(ones_t, deg_s.at[dst_t], add=True)


def _sage_agg1_kernel():
    @functools.partial(
        pl.kernel,
        out_type=(
            jax.ShapeDtypeStruct((N, D), jnp.float32),
            jax.ShapeDtypeStruct((NC * N,), jnp.float32),
            jax.ShapeDtypeStruct((NC, N, D), jnp.float32),
        ),
        mesh=_MESH,
        scratch_types=[
            pltpu.VMEM((EB,), jnp.int32), pltpu.VMEM((EB,), jnp.int32),
            pltpu.VMEM((EB,), jnp.int32), pltpu.VMEM((EB,), jnp.int32),
            pltpu.VMEM((EB,), jnp.int32), pltpu.VMEM((EB,), jnp.int32),
            pltpu.VMEM((EB, D), jnp.float32), pltpu.VMEM((EB, D), jnp.float32),
            pltpu.VMEM((EB,), jnp.float32),
            pltpu.VMEM((ETAIL,), jnp.int32),
            pltpu.VMEM((ETAIL,), jnp.int32),
            pltpu.VMEM((ETAIL,), jnp.int32),
            pltpu.VMEM((ETAIL, D), jnp.float32),
            pltpu.VMEM((ETAIL,), jnp.float32),
            pltpu.VMEM((N,), jnp.float32),
            pltpu.VMEM_SHARED((N, D), jnp.float32),
            pltpu.VMEM_SHARED((N,), jnp.float32),
            pltpu.SemaphoreType.DMA, pltpu.SemaphoreType.DMA,
            pltpu.SemaphoreType.DMA, pltpu.SemaphoreType.DMA,
            pltpu.SemaphoreType.DMA, pltpu.SemaphoreType.DMA,
            pltpu.SemaphoreType.DMA, pltpu.SemaphoreType.DMA,
            pltpu.SemaphoreType.DMA, pltpu.SemaphoreType.DMA,
            pltpu.SemaphoreType.DMA, pltpu.SemaphoreType.DMA,
            pltpu.SemaphoreType.DMA,
        ],
    )
    def k(x_h, n_id_h, src_h, dst_h, zrows_h, zdeg_h,
          h0_h, deg_h, agg_h,
          idx0, idx1, gidx0, gidx1, dst0, dst1, rows0, rows1, ones_v,
          idx_t, gidx_t, dst_t, rows_t, ones_t, deg_v,
          agg_s, deg_s,
          si0, si1, sd0, sd1, sr0, sr1, sg0, sg1, ss0, ss1, sq0, sq1, sem):
        c = lax.axis_index("c")
        s = lax.axis_index("s")
        wid = c * NS + s

        pltpu.sync_copy(zrows_h.at[pl.ds(s * RPS, RPS)], agg_s.at[pl.ds(s * RPS, RPS)])

        @pl.when(s == NS - 1)
        def _():
            pltpu.sync_copy(zrows_h.at[pl.ds(NS * RPS, RTAIL)],
                            agg_s.at[pl.ds(NS * RPS, RTAIL)])

        @pl.when(s == 0)
        def _():
            pltpu.sync_copy(zdeg_h, deg_v)
            pltpu.sync_copy(deg_v, deg_s)

        _fill_ones(ones_v, EB)
        _fill_ones(ones_t, ETAIL)
        plsc.subcore_barrier()

        _edge_pipeline(x_h, n_id_h, src_h, dst_h, wid,
                       [idx0, idx1], [gidx0, gidx1], [dst0, dst1],
                       [rows0, rows1], ones_v,
                       [si0, si1], [sd0, sd1], [sr0, sr1], [sg0, sg1],
                       [ss0, ss1], [sq0, sq1],
                       agg_s, deg_s, compose=True)
        _edge_tail(x_h, n_id_h, src_h, dst_h, wid * EPT + NFULL * EB,
                   idx_t, gidx_t, dst_t, rows_t, ones_t, sem,
                   agg_s, deg_s, compose=True)

        for kk in range((NGB + NW - 1) // NW):
            b = wid + NW * kk

            @pl.when(b < NGB)
            def _():
                pltpu.sync_copy(n_id_h.at[pl.ds(b * GB, GB)], idx0)
                pltpu.async_copy(x_h.at[idx0], rows0, sem).wait()
                pltpu.sync_copy(rows0, h0_h.at[pl.ds(b * GB, GB)])

        @pl.when(wid == 0)
        def _():
            pltpu.sync_copy(n_id_h.at[pl.ds(NGB * GB, GTAIL)], idx_t)
            pltpu.async_copy(x_h.at[idx_t], rows_t, sem).wait()
            pltpu.sync_copy(rows_t, h0_h.at[pl.ds(NGB * GB, GTAIL)])

        plsc.subcore_barrier()

        pltpu.sync_copy(agg_s.at[pl.ds(s * RPS, RPS)], agg_h.at[c, pl.ds(s * RPS, RPS)])

        @pl.when(s == NS - 1)
        def _():
            pltpu.sync_copy(agg_s.at[pl.ds(NS * RPS, RTAIL)],
                            agg_h.at[c, pl.ds(NS * RPS, RTAIL)])

        @pl.when(s == 0)
        def _():
            pltpu.sync_copy(deg_s, deg_v)
            pltpu.sync_copy(deg_v, deg_h.at[pl.ds(c * N, N)])

    return k


def _sage_agg2_kernel():
    @functools.partial(
        pl.kernel,
        out_type=jax.ShapeDtypeStruct((NC, N, D), jnp.float32),
        mesh=_MESH,
        scratch_types=[
            pltpu.VMEM((EB,), jnp.int32), pltpu.VMEM((EB,), jnp.int32),
            pltpu.VMEM((EB,), jnp.int32), pltpu.VMEM((EB,), jnp.int32),
            pltpu.VMEM((EB, D), jnp.float32), pltpu.VMEM((EB, D), jnp.float32),
            pltpu.VMEM((ETAIL,), jnp.int32),
            pltpu.VMEM((ETAIL,), jnp.int32),
            pltpu.VMEM((ETAIL, D), jnp.float32),
            pltpu.VMEM_SHARED((N, D), jnp.float32),
            pltpu.SemaphoreType.DMA, pltpu.SemaphoreType.DMA,
            pltpu.SemaphoreType.DMA, pltpu.SemaphoreType.DMA,
            pltpu.SemaphoreType.DMA, pltpu.SemaphoreType.DMA,
            pltpu.SemaphoreType.DMA, pltpu.SemaphoreType.DMA,
            pltpu.SemaphoreType.DMA,
        ],
    )
    def k(h1_h, src_h, dst_h, zrows_h, agg_h,
          idx0, idx1, dst0, dst1, rows0, rows1,
          idx_t, dst_t, rows_t, agg_s,
          si0, si1, sd0, sd1, sr0, sr1, ss0, ss1, sem):
        c = lax.axis_index("c")
        s = lax.axis_index("s")
        wid = c * NS + s

        pltpu.sync_copy(zrows_h.at[pl.ds(s * RPS, RPS)], agg_s.at[pl.ds(s * RPS, RPS)])

        @pl.when(s == NS - 1)
        def _():
            pltpu.sync_copy(zrows_h.at[pl.ds(NS * RPS, RTAIL)],
                            agg_s.at[pl.ds(NS * RPS, RTAIL)])

        plsc.subcore_barrier()

        _edge_pipeline(h1_h, None, src_h, dst_h, wid,
                       [idx0, idx1], None, [dst0, dst1],
                       [rows0, rows1], None,
                       [si0, si1], [sd0, sd1], [sr0, sr1], None,
                       [ss0, ss1], None,
                       agg_s, None, compose=False)
        _edge_tail(h1_h, None, src_h, dst_h, wid * EPT + NFULL * EB,
                   idx_t, None, dst_t, rows_t, None, sem,
                   agg_s, None, compose=False)

        plsc.subcore_barrier()
        pltpu.sync_copy(agg_s.at[pl.ds(s * RPS, RPS)], agg_h.at[c, pl.ds(s * RPS, RPS)])

        @pl.when(s == NS - 1)
        def _():
            pltpu.sync_copy(agg_s.at[pl.ds(NS * RPS, RTAIL)],
                            agg_h.at[c, pl.ds(NS * RPS, RTAIL)])

    return k


_R = 1000


def _tc1_body(h0, aggp, degp, ws, wn, b, out):
    agg = aggp[0] + aggp[1]
    deg = degp[0] + degp[1]
    mean = agg / jnp.maximum(deg, 1.0)
    acc = jnp.dot(h0[...], ws[...], preferred_element_type=jnp.float32)
    acc += jnp.dot(mean, wn[...], preferred_element_type=jnp.float32)
    out[...] = jnp.maximum(acc + b[...], 0.0)


def _tc2_body(h1, aggp, degp, w2s, w2n, b2, wl1, bl1, wl2, bl2, out):
    agg = aggp[0] + aggp[1]
    deg = degp[0] + degp[1]
    mean = agg / jnp.maximum(deg, 1.0)
    h2 = jnp.dot(h1[...], w2s[...], preferred_element_type=jnp.float32)
    h2 += jnp.dot(mean, w2n[...], preferred_element_type=jnp.float32)
    h2 = jnp.maximum(h2 + b2[...], 0.0)
    h3 = jnp.maximum(
        jnp.dot(h2, wl1[...], preferred_element_type=jnp.float32) + bl1[...], 0.0)
    logits = jnp.dot(h3, wl2[...], preferred_element_type=jnp.float32) + bl2[...]
    m = jnp.max(logits, axis=-1, keepdims=True)
    lse = jnp.log(jnp.sum(jnp.exp(logits - m), axis=-1, keepdims=True)) + m
    out[...] = logits - lse


def _row_specs():
    rows = pl.BlockSpec((_R, D), lambda i: (i, 0))
    aggp = pl.BlockSpec((NC, _R, D), lambda i: (0, i, 0))
    degp = pl.BlockSpec((NC, _R, 1), lambda i: (0, i, 0))
    w = pl.BlockSpec((D, D), lambda i: (0, 0))
    bias = pl.BlockSpec((1, D), lambda i: (0, 0))
    return rows, aggp, degp, w, bias


def _tc1_call(h0, agg, deg, ws, wn, b):
    rows, aggp, degp, w, bias = _row_specs()
    return pl.pallas_call(
        _tc1_body,
        grid=(N // _R,),
        in_specs=[rows, aggp, degp, w, w, bias],
        out_specs=rows,
        out_shape=jax.ShapeDtypeStruct((N, H), jnp.float32),
    )(h0, agg, deg.reshape(NC, N, 1), ws, wn, b.reshape(1, H))


def _tc2_call(h1, agg, deg, w2s, w2n, b2, wl1, bl1, wl2, bl2):
    rows, aggp, degp, w, bias = _row_specs()
    wc = pl.BlockSpec((H, C), lambda i: (0, 0))
    bc = pl.BlockSpec((1, C), lambda i: (0, 0))
    outc = pl.BlockSpec((_R, C), lambda i: (i, 0))
    return pl.pallas_call(
        _tc2_body,
        grid=(N // _R,),
        in_specs=[rows, aggp, degp, w, w, bias, w, bias, wc, bc],
        out_specs=outc,
        out_shape=jax.ShapeDtypeStruct((N, C), jnp.float32),
    )(h1, agg, deg.reshape(NC, N, 1), w2s, w2n, b2.reshape(1, H),
      wl1, bl1.reshape(1, H), wl2, bl2.reshape(1, C))


def kernel(x, W1_self, W1_neigh, b1, W2_self, W2_neigh, b2,
           W_lin1, b_lin1, W_lin2, b_lin2, n_id, edge_index):
    src = edge_index[0]
    dst = edge_index[1]
    zrows = jnp.zeros((N, D), jnp.float32)
    zdeg = jnp.zeros((N,), jnp.float32)

    h0, deg, agg1 = _sage_agg1_kernel()(x, n_id, src, dst, zrows, zdeg)
    deg = deg.reshape(NC, N)
    h1 = _tc1_call(h0, agg1, deg, W1_self, W1_neigh, b1)
    agg2 = _sage_agg2_kernel()(h1, src, dst, zrows)
    return _tc2_call(h1, agg2, deg, W2_self, W2_neigh, b2,
                     W_lin1, b_lin1, W_lin2, b_lin2)


# speedup vs baseline: 11.4853x; 1.2265x over previous
import functools

import jax
import jax.numpy as jnp
from jax import lax
from jax.experimental import pallas as pl
from jax.experimental.pallas import tpu as pltpu
from jax.experimental.pallas import tpu_sc as plsc

N = 10000
E = 320000
D = 128
H = 128
C = 64

NC = 2
NS = 16
NW = NC * NS
EPT = E // NW
EB = 128
NFULL = EPT // EB
ETAIL = EPT - NFULL * EB
RPS = 624
RTAIL = N - NS * RPS
GB = 128
NGB = N // GB
GTAIL = N - NGB * GB

_MESH = plsc.VectorSubcoreMesh(
    core_axis_name="c", subcore_axis_name="s", num_cores=NC, num_subcores=NS)


def _fill_ones(ref, n):
    for i in range(n // 16):
        ref[pl.ds(i * 16, 16)] = jnp.full((16,), 1.0, jnp.float32)


CK = 13
NCH = NFULL // CK


def _make_agg_kernel(layer1):
    if layer1:
        out_type = (
            jax.ShapeDtypeStruct((N, D), jnp.float32),
            jax.ShapeDtypeStruct((NC * N,), jnp.float32),
            jax.ShapeDtypeStruct((NC, N, D), jnp.float32),
        )
    else:
        out_type = jax.ShapeDtypeStruct((NC, N, D), jnp.float32)

    scratch = [
        pltpu.VMEM((EPT,), jnp.int32),
        pltpu.VMEM((EB,), jnp.int32),
        pltpu.VMEM((EB,), jnp.int32),
        pltpu.VMEM((ETAIL,), jnp.int32),
        pltpu.VMEM((EB, D), jnp.float32),
        pltpu.VMEM((EB, D), jnp.float32),
        pltpu.VMEM_SHARED((N, D), jnp.float32),
        pltpu.SemaphoreType.DMA,
        pltpu.SemaphoreType.DMA,
        pltpu.SemaphoreType.DMA,
        pltpu.SemaphoreType.DMA,
        pltpu.SemaphoreType.DMA,
        pltpu.SemaphoreType.DMA,
        pltpu.SemaphoreType.DMA,
        pltpu.SemaphoreType.DMA,
    ]
    if layer1:
        scratch += [
            pltpu.VMEM((CK * EB,), jnp.int32),
            pltpu.VMEM((CK * EB,), jnp.int32),
            pltpu.VMEM((EB,), jnp.float32),
            pltpu.VMEM((ETAIL,), jnp.float32),
            pltpu.VMEM((RPS,), jnp.float32),
            pltpu.VMEM((RTAIL,), jnp.float32),
            pltpu.VMEM_SHARED((N,), jnp.float32),
            pltpu.SemaphoreType.DMA,
            pltpu.SemaphoreType.DMA,
            pltpu.SemaphoreType.DMA,
            pltpu.SemaphoreType.DMA,
            pltpu.SemaphoreType.DMA,
        ]

    @functools.partial(pl.kernel, out_type=out_type, mesh=_MESH,
                       scratch_types=scratch)
    def k(*refs):
        if layer1:
            (table, n_id_h, src_h, dst_h, zrows_h, zdeg_h,
             h0_h, deg_h, agg_h,
             idx_all, dstb0, dstb1, dst_t, rows0, rows1, agg_s,
             s_idx, s_d0, s_d1, s_r0, s_r1, s_s0, s_s1, sem,
             srcb0, srcb1, ones_v, ones_t, deg_v, deg_t, deg_s,
             s_gid, s_b0, s_b1, s_q0, s_q1) = refs
        else:
            (table, src_h, dst_h, zrows_h,
             agg_h,
             idx_all, dstb0, dstb1, dst_t, rows0, rows1, agg_s,
             s_idx, s_d0, s_d1, s_r0, s_r1, s_s0, s_s1, sem) = refs

        c = lax.axis_index("c")
        s = lax.axis_index("s")
        wid = c * NS + s
        ebase = wid * EPT
        rows = [rows0, rows1]
        dstb = [dstb0, dstb1]
        s_r = [s_r0, s_r1]
        s_s = [s_s0, s_s1]
        s_d = [s_d0, s_d1]

        if not layer1:
            pltpu.async_copy(src_h.at[pl.ds(ebase, EPT)], idx_all, s_idx)
        else:
            srcb = [srcb0, srcb1]
            s_b = [s_b0, s_b1]

            def fire_ld(ch, b):
                pltpu.async_copy(src_h.at[pl.ds(ebase + ch * CK * EB, CK * EB)],
                                 srcb[b], s_b[b])

            def wait_ld(ch, b):
                pltpu.make_async_copy(
                    src_h.at[pl.ds(ebase + ch * CK * EB, CK * EB)],
                    srcb[b], s_b[b]).wait()

            fire_ld(0, 0)

        pltpu.async_copy(dst_h.at[pl.ds(ebase + NFULL * EB, ETAIL)], dst_t, s_d0)

        pltpu.sync_copy(zrows_h.at[pl.ds(s * RPS, RPS)],
                        agg_s.at[pl.ds(s * RPS, RPS)])

        @pl.when(s == NS - 1)
        def _():
            pltpu.sync_copy(zrows_h.at[pl.ds(NS * RPS, RTAIL)],
                            agg_s.at[pl.ds(NS * RPS, RTAIL)])

        if layer1:
            pltpu.sync_copy(zdeg_h.at[pl.ds(s * RPS, RPS)], deg_v)
            pltpu.sync_copy(deg_v, deg_s.at[pl.ds(s * RPS, RPS)])

            @pl.when(s == NS - 1)
            def _():
                pltpu.sync_copy(zdeg_h.at[pl.ds(NS * RPS, RTAIL)], deg_t)
                pltpu.sync_copy(deg_t, deg_s.at[pl.ds(NS * RPS, RTAIL)])

            _fill_ones(ones_v, EB)
            _fill_ones(ones_t, ETAIL)

            for ch in range(NCH):
                b = ch % 2
                wait_ld(ch, b)
                if ch + 1 < NCH:
                    fire_ld(ch + 1, 1 - b)
                for j in range(CK):
                    i = ch * CK + j
                    pltpu.async_copy(
                        n_id_h.at[srcb[b].at[pl.ds(j * EB, EB)]],
                        idx_all.at[pl.ds(i * EB, EB)], s_gid)
                for j in range(CK):
                    i = ch * CK + j
                    pltpu.make_async_copy(
                        n_id_h.at[srcb[b].at[pl.ds(j * EB, EB)]],
                        idx_all.at[pl.ds(i * EB, EB)], s_gid).wait()
            pltpu.sync_copy(src_h.at[pl.ds(ebase + NFULL * EB, ETAIL)],
                            srcb0.at[pl.ds(0, ETAIL)])
            pltpu.async_copy(
                n_id_h.at[srcb0.at[pl.ds(0, ETAIL)]],
                idx_all.at[pl.ds(NFULL * EB, ETAIL)], s_gid).wait()

        plsc.subcore_barrier()

        if not layer1:
            pltpu.make_async_copy(src_h.at[pl.ds(ebase, EPT)], idx_all,
                                  s_idx).wait()

        def fire_d(i, b):
            pltpu.async_copy(dst_h.at[pl.ds(ebase + i * EB, EB)],
                             dstb[b], s_d[b])

        def wait_d(i, b):
            pltpu.make_async_copy(dst_h.at[pl.ds(ebase + i * EB, EB)],
                                  dstb[b], s_d[b]).wait()

        def fire_g(i, b):
            pltpu.async_copy(table.at[idx_all.at[pl.ds(i * EB, EB)]],
                             rows[b], s_r[b])

        def wait_g(i, b):
            pltpu.make_async_copy(table.at[idx_all.at[pl.ds(i * EB, EB)]],
                                  rows[b], s_r[b]).wait()

        def fire_s(i, b):
            pltpu.async_copy(rows[b], agg_s.at[dstb[b]], s_s[b], add=True)
            if layer1:
                pltpu.async_copy(ones_v, deg_s.at[dstb[b]],
                                 [s_q0, s_q1][b], add=True)

        def wait_s(i, b):
            pltpu.make_async_copy(rows[b], agg_s.at[dstb[b]], s_s[b]).wait()
            if layer1:
                pltpu.make_async_copy(ones_v, deg_s.at[dstb[b]],
                                      [s_q0, s_q1][b]).wait()

        def fire_batch(i, b):
            fire_d(i, b)
            fire_g(i, b)

        fire_batch(0, 0)

        def grp(g, carry):
            i = 2 * g
            pl.when(g >= 1)(lambda: wait_s(i - 1, 1))
            fire_batch(i + 1, 1)
            wait_g(i, 0)
            wait_d(i, 0)
            fire_s(i, 0)
            i = 2 * g + 1
            wait_s(i - 1, 0)
            pl.when(g < NFULL // 2 - 1)(lambda: fire_batch(i + 1, 0))
            wait_g(i, 1)
            wait_d(i, 1)
            fire_s(i, 1)
            return carry

        lax.fori_loop(0, NFULL // 2, grp, 0)
        wait_s(NFULL - 1, 1)

        pltpu.make_async_copy(dst_h.at[pl.ds(ebase + NFULL * EB, ETAIL)],
                              dst_t, s_d0).wait()
        pltpu.async_copy(table.at[idx_all.at[pl.ds(NFULL * EB, ETAIL)]],
                         rows0.at[pl.ds(0, ETAIL), :], sem).wait()
        pltpu.sync_copy(rows0.at[pl.ds(0, ETAIL), :], agg_s.at[dst_t], add=True)
        if layer1:
            pltpu.sync_copy(ones_t, deg_s.at[dst_t], add=True)

        if layer1:
            for kk in range((NGB + NW - 1) // NW):
                b = wid + NW * kk

                @pl.when(b < NGB)
                def _():
                    pltpu.sync_copy(n_id_h.at[pl.ds(b * GB, GB)],
                                    srcb0.at[pl.ds(0, GB)])
                    pltpu.async_copy(table.at[srcb0.at[pl.ds(0, GB)]],
                                     rows1, sem).wait()
                    pltpu.sync_copy(rows1, h0_h.at[pl.ds(b * GB, GB)])

            @pl.when(wid == 0)
            def _():
                pltpu.sync_copy(n_id_h.at[pl.ds(NGB * GB, GTAIL)],
                                srcb1.at[pl.ds(0, GTAIL)])
                pltpu.async_copy(table.at[srcb1.at[pl.ds(0, GTAIL)]],
                                 rows0.at[pl.ds(0, GTAIL), :], sem).wait()
                pltpu.sync_copy(rows0.at[pl.ds(0, GTAIL), :],
                                h0_h.at[pl.ds(NGB * GB, GTAIL)])

        plsc.subcore_barrier()

        pltpu.sync_copy(agg_s.at[pl.ds(s * RPS, RPS)],
                        agg_h.at[c, pl.ds(s * RPS, RPS)])

        @pl.when(s == NS - 1)
        def _():
            pltpu.sync_copy(agg_s.at[pl.ds(NS * RPS, RTAIL)],
                            agg_h.at[c, pl.ds(NS * RPS, RTAIL)])

        if layer1:
            pltpu.sync_copy(deg_s.at[pl.ds(s * RPS, RPS)], deg_v)
            pltpu.sync_copy(deg_v, deg_h.at[pl.ds(c * N + s * RPS, RPS)])

            @pl.when(s == NS - 1)
            def _():
                pltpu.sync_copy(deg_s.at[pl.ds(NS * RPS, RTAIL)], deg_t)
                pltpu.sync_copy(deg_t, deg_h.at[pl.ds(c * N + NS * RPS, RTAIL)])

    return k


_R = 1000


def _tc1_body(h0, aggp, degp, ws, wn, b, out):
    agg = aggp[0] + aggp[1]
    deg = degp[0] + degp[1]
    mean = agg / jnp.maximum(deg, 1.0)
    acc = jnp.dot(h0[...], ws[...], preferred_element_type=jnp.float32)
    acc += jnp.dot(mean, wn[...], preferred_element_type=jnp.float32)
    out[...] = jnp.maximum(acc + b[...], 0.0)


def _tc2_body(h1, aggp, degp, w2s, w2n, b2, wl1, bl1, wl2, bl2, out):
    agg = aggp[0] + aggp[1]
    deg = degp[0] + degp[1]
    mean = agg / jnp.maximum(deg, 1.0)
    h2 = jnp.dot(h1[...], w2s[...], preferred_element_type=jnp.float32)
    h2 += jnp.dot(mean, w2n[...], preferred_element_type=jnp.float32)
    h2 = jnp.maximum(h2 + b2[...], 0.0)
    h3 = jnp.maximum(
        jnp.dot(h2, wl1[...], preferred_element_type=jnp.float32) + bl1[...], 0.0)
    logits = jnp.dot(h3, wl2[...], preferred_element_type=jnp.float32) + bl2[...]
    m = jnp.max(logits, axis=-1, keepdims=True)
    lse = jnp.log(jnp.sum(jnp.exp(logits - m), axis=-1, keepdims=True)) + m
    out[...] = logits - lse


def _row_specs():
    rows = pl.BlockSpec((_R, D), lambda i: (i, 0))
    aggp = pl.BlockSpec((NC, _R, D), lambda i: (0, i, 0))
    degp = pl.BlockSpec((NC, _R, 1), lambda i: (0, i, 0))
    w = pl.BlockSpec((D, D), lambda i: (0, 0))
    bias = pl.BlockSpec((1, D), lambda i: (0, 0))
    return rows, aggp, degp, w, bias


def _tc1_call(h0, agg, deg, ws, wn, b):
    rows, aggp, degp, w, bias = _row_specs()
    return pl.pallas_call(
        _tc1_body,
        grid=(N // _R,),
        in_specs=[rows, aggp, degp, w, w, bias],
        out_specs=rows,
        out_shape=jax.ShapeDtypeStruct((N, H), jnp.float32),
    )(h0, agg, deg.reshape(NC, N, 1), ws, wn, b.reshape(1, H))


def _tc2_call(h1, agg, deg, w2s, w2n, b2, wl1, bl1, wl2, bl2):
    rows, aggp, degp, w, bias = _row_specs()
    wc = pl.BlockSpec((H, C), lambda i: (0, 0))
    bc = pl.BlockSpec((1, C), lambda i: (0, 0))
    outc = pl.BlockSpec((_R, C), lambda i: (i, 0))
    return pl.pallas_call(
        _tc2_body,
        grid=(N // _R,),
        in_specs=[rows, aggp, degp, w, w, bias, w, bias, wc, bc],
        out_specs=outc,
        out_shape=jax.ShapeDtypeStruct((N, C), jnp.float32),
    )(h1, agg, deg.reshape(NC, N, 1), w2s, w2n, b2.reshape(1, H),
      wl1, bl1.reshape(1, H), wl2, bl2.reshape(1, C))


def kernel(x, W1_self, W1_neigh, b1, W2_self, W2_neigh, b2,
           W_lin1, b_lin1, W_lin2, b_lin2, n_id, edge_index):
    src = edge_index[0]
    dst = edge_index[1]
    zrows = jnp.zeros((N, D), jnp.float32)
    zdeg = jnp.zeros((N,), jnp.float32)

    h0, deg, agg1 = _make_agg_kernel(True)(x, n_id, src, dst, zrows, zdeg)
    deg = deg.reshape(NC, N)
    h1 = _tc1_call(h0, agg1, deg, W1_self, W1_neigh, b1)
    agg2 = _make_agg_kernel(False)(h1, src, dst, zrows)
    return _tc2_call(h1, agg2, deg, W2_self, W2_neigh, b2,
                     W_lin1, b_lin1, W_lin2, b_lin2)


# speedup vs baseline: 12.4529x; 1.0842x over previous
import functools

import jax
import jax.numpy as jnp
from jax import lax
from jax.experimental import pallas as pl
from jax.experimental.pallas import tpu as pltpu
from jax.experimental.pallas import tpu_sc as plsc

N = 10000
E = 320000
D = 128
H = 128
C = 64

NC = 2
NS = 16
NW = NC * NS
EPT = E // NW
EB = 128
NFULL = EPT // EB
ETAIL = EPT - NFULL * EB
RPS = 624
RTAIL = N - NS * RPS
GB = 128
NGB = N // GB
GTAIL = N - NGB * GB

_MESH = plsc.VectorSubcoreMesh(
    core_axis_name="c", subcore_axis_name="s", num_cores=NC, num_subcores=NS)


def _fill_ones(ref, n):
    for i in range(n // 16):
        ref[pl.ds(i * 16, 16)] = jnp.full((16,), 1.0, jnp.float32)


H0B = 5


def _make_agg_kernel(layer1):
    if layer1:
        out_type = (
            jax.ShapeDtypeStruct((N, D), jnp.float32),
            jax.ShapeDtypeStruct((NC * N,), jnp.float32),
            jax.ShapeDtypeStruct((NC, N, D), jnp.float32),
        )
    else:
        out_type = jax.ShapeDtypeStruct((NC, N, D), jnp.float32)

    scratch = [
        pltpu.VMEM((EPT,), jnp.int32),
        pltpu.VMEM((EB,), jnp.int32),
        pltpu.VMEM((EB,), jnp.int32),
        pltpu.VMEM((ETAIL,), jnp.int32),
        pltpu.VMEM((EB, D), jnp.float32),
        pltpu.VMEM((EB, D), jnp.float32),
        pltpu.VMEM_SHARED((N, D), jnp.float32),
        pltpu.SemaphoreType.DMA,
        pltpu.SemaphoreType.DMA,
        pltpu.SemaphoreType.DMA,
        pltpu.SemaphoreType.DMA,
        pltpu.SemaphoreType.DMA,
        pltpu.SemaphoreType.DMA,
        pltpu.SemaphoreType.DMA,
        pltpu.SemaphoreType.DMA,
    ]
    if layer1:
        scratch += [
            pltpu.VMEM((H0B * GB,), jnp.int32),
            pltpu.VMEM((EB,), jnp.float32),
            pltpu.VMEM((ETAIL,), jnp.float32),
            pltpu.VMEM((RPS,), jnp.float32),
            pltpu.VMEM((RTAIL,), jnp.float32),
            pltpu.VMEM_SHARED((N,), jnp.float32),
            pltpu.SemaphoreType.DMA,
            pltpu.SemaphoreType.DMA,
            pltpu.SemaphoreType.DMA,
        ]

    @functools.partial(pl.kernel, out_type=out_type, mesh=_MESH,
                       scratch_types=scratch)
    def k(*refs):
        if layer1:
            (x_h, n_id_h, src_h, dst_h, zrows_h, zdeg_h,
             h0_h, deg_h, agg_h,
             idx_all, dstb0, dstb1, dst_t, rows0, rows1, agg_s,
             s_idx, s_d0, s_d1, s_r0, s_r1, s_s0, s_s1, sem,
             idx_h0, ones_v, ones_t, deg_v, deg_t, deg_s,
             s_h0, s_q0, s_q1) = refs
            table = h0_h
        else:
            (table, src_h, dst_h, zrows_h,
             agg_h,
             idx_all, dstb0, dstb1, dst_t, rows0, rows1, agg_s,
             s_idx, s_d0, s_d1, s_r0, s_r1, s_s0, s_s1, sem) = refs

        c = lax.axis_index("c")
        s = lax.axis_index("s")
        wid = c * NS + s
        ebase = wid * EPT
        rows = [rows0, rows1]
        dstb = [dstb0, dstb1]
        s_r = [s_r0, s_r1]
        s_s = [s_s0, s_s1]
        s_d = [s_d0, s_d1]

        pltpu.async_copy(src_h.at[pl.ds(ebase, EPT)], idx_all, s_idx)
        pltpu.async_copy(dst_h.at[pl.ds(ebase + NFULL * EB, ETAIL)], dst_t, s_d0)
        if layer1:
            @pl.when(s < NS - 1)
            def _():
                pltpu.async_copy(n_id_h.at[pl.ds(s * H0B * GB, H0B * GB)],
                                 idx_h0, s_h0)

            @pl.when(s == NS - 1)
            def _():
                pltpu.async_copy(
                    n_id_h.at[pl.ds((NS - 1) * H0B * GB, N - (NS - 1) * H0B * GB)],
                    idx_h0.at[pl.ds(0, N - (NS - 1) * H0B * GB)], s_h0)

        pltpu.sync_copy(zrows_h.at[pl.ds(s * RPS, RPS)],
                        agg_s.at[pl.ds(s * RPS, RPS)])

        @pl.when(s == NS - 1)
        def _():
            pltpu.sync_copy(zrows_h.at[pl.ds(NS * RPS, RTAIL)],
                            agg_s.at[pl.ds(NS * RPS, RTAIL)])

        if layer1:
            pltpu.sync_copy(zdeg_h.at[pl.ds(s * RPS, RPS)], deg_v)
            pltpu.sync_copy(deg_v, deg_s.at[pl.ds(s * RPS, RPS)])

            @pl.when(s == NS - 1)
            def _():
                pltpu.sync_copy(zdeg_h.at[pl.ds(NS * RPS, RTAIL)], deg_t)
                pltpu.sync_copy(deg_t, deg_s.at[pl.ds(NS * RPS, RTAIL)])

            _fill_ones(ones_v, EB)
            _fill_ones(ones_t, ETAIL)

            @pl.when(s < NS - 1)
            def _():
                pltpu.make_async_copy(
                    n_id_h.at[pl.ds(s * H0B * GB, H0B * GB)], idx_h0,
                    s_h0).wait()

            @pl.when(s == NS - 1)
            def _():
                pltpu.make_async_copy(
                    n_id_h.at[pl.ds((NS - 1) * H0B * GB, N - (NS - 1) * H0B * GB)],
                    idx_h0.at[pl.ds(0, N - (NS - 1) * H0B * GB)], s_h0).wait()

            for kk in range(H0B):
                b = s * H0B + kk

                @pl.when(b < NGB)
                def _():
                    pltpu.async_copy(
                        x_h.at[idx_h0.at[pl.ds(kk * GB, GB)]],
                        rows[kk % 2], s_r[kk % 2]).wait()
                    pltpu.sync_copy(rows[kk % 2], h0_h.at[pl.ds(b * GB, GB)])

            @pl.when(s == NS - 1)
            def _():
                pltpu.async_copy(
                    x_h.at[idx_h0.at[pl.ds((NGB - (NS - 1) * H0B) * GB, GTAIL)]],
                    rows0.at[pl.ds(0, GTAIL), :], sem).wait()
                pltpu.sync_copy(rows0.at[pl.ds(0, GTAIL), :],
                                h0_h.at[pl.ds(NGB * GB, GTAIL)])

        plsc.subcore_barrier()

        pltpu.make_async_copy(src_h.at[pl.ds(ebase, EPT)], idx_all,
                              s_idx).wait()

        def fire_d(i, b):
            pltpu.async_copy(dst_h.at[pl.ds(ebase + i * EB, EB)],
                             dstb[b], s_d[b])

        def wait_d(i, b):
            pltpu.make_async_copy(dst_h.at[pl.ds(ebase + i * EB, EB)],
                                  dstb[b], s_d[b]).wait()

        def fire_g(i, b):
            pltpu.async_copy(table.at[idx_all.at[pl.ds(i * EB, EB)]],
                             rows[b], s_r[b])

        def wait_g(i, b):
            pltpu.make_async_copy(table.at[idx_all.at[pl.ds(i * EB, EB)]],
                                  rows[b], s_r[b]).wait()

        def fire_s(i, b):
            pltpu.async_copy(rows[b], agg_s.at[dstb[b]], s_s[b], add=True)
            if layer1:
                pltpu.async_copy(ones_v, deg_s.at[dstb[b]],
                                 [s_q0, s_q1][b], add=True)

        def wait_s(i, b):
            pltpu.make_async_copy(rows[b], agg_s.at[dstb[b]], s_s[b]).wait()
            if layer1:
                pltpu.make_async_copy(ones_v, deg_s.at[dstb[b]],
                                      [s_q0, s_q1][b]).wait()

        def fire_batch(i, b):
            fire_d(i, b)
            fire_g(i, b)

        fire_batch(0, 0)

        def grp(g, carry):
            i = 2 * g
            pl.when(g >= 1)(lambda: wait_s(i - 1, 1))
            fire_batch(i + 1, 1)
            wait_g(i, 0)
            wait_d(i, 0)
            fire_s(i, 0)
            i = 2 * g + 1
            wait_s(i - 1, 0)
            pl.when(g < NFULL // 2 - 1)(lambda: fire_batch(i + 1, 0))
            wait_g(i, 1)
            wait_d(i, 1)
            fire_s(i, 1)
            return carry

        lax.fori_loop(0, NFULL // 2, grp, 0)
        wait_s(NFULL - 1, 1)

        pltpu.make_async_copy(dst_h.at[pl.ds(ebase + NFULL * EB, ETAIL)],
                              dst_t, s_d0).wait()
        pltpu.async_copy(table.at[idx_all.at[pl.ds(NFULL * EB, ETAIL)]],
                         rows0.at[pl.ds(0, ETAIL), :], sem).wait()
        pltpu.sync_copy(rows0.at[pl.ds(0, ETAIL), :], agg_s.at[dst_t], add=True)
        if layer1:
            pltpu.sync_copy(ones_t, deg_s.at[dst_t], add=True)

        plsc.subcore_barrier()

        pltpu.sync_copy(agg_s.at[pl.ds(s * RPS, RPS)],
                        agg_h.at[c, pl.ds(s * RPS, RPS)])

        @pl.when(s == NS - 1)
        def _():
            pltpu.sync_copy(agg_s.at[pl.ds(NS * RPS, RTAIL)],
                            agg_h.at[c, pl.ds(NS * RPS, RTAIL)])

        if layer1:
            pltpu.sync_copy(deg_s.at[pl.ds(s * RPS, RPS)], deg_v)
            pltpu.sync_copy(deg_v, deg_h.at[pl.ds(c * N + s * RPS, RPS)])

            @pl.when(s == NS - 1)
            def _():
                pltpu.sync_copy(deg_s.at[pl.ds(NS * RPS, RTAIL)], deg_t)
                pltpu.sync_copy(deg_t, deg_h.at[pl.ds(c * N + NS * RPS, RTAIL)])

    return k


_R = 1000


def _tc1_body(h0, aggp, degp, ws, wn, b, out):
    agg = aggp[0] + aggp[1]
    deg = degp[0] + degp[1]
    mean = agg / jnp.maximum(deg, 1.0)
    acc = jnp.dot(h0[...], ws[...], preferred_element_type=jnp.float32)
    acc += jnp.dot(mean, wn[...], preferred_element_type=jnp.float32)
    out[...] = jnp.maximum(acc + b[...], 0.0)


def _tc2_body(h1, aggp, degp, w2s, w2n, b2, wl1, bl1, wl2, bl2, out):
    agg = aggp[0] + aggp[1]
    deg = degp[0] + degp[1]
    mean = agg / jnp.maximum(deg, 1.0)
    h2 = jnp.dot(h1[...], w2s[...], preferred_element_type=jnp.float32)
    h2 += jnp.dot(mean, w2n[...], preferred_element_type=jnp.float32)
    h2 = jnp.maximum(h2 + b2[...], 0.0)
    h3 = jnp.maximum(
        jnp.dot(h2, wl1[...], preferred_element_type=jnp.float32) + bl1[...], 0.0)
    logits = jnp.dot(h3, wl2[...], preferred_element_type=jnp.float32) + bl2[...]
    m = jnp.max(logits, axis=-1, keepdims=True)
    lse = jnp.log(jnp.sum(jnp.exp(logits - m), axis=-1, keepdims=True)) + m
    out[...] = logits - lse


def _row_specs():
    rows = pl.BlockSpec((_R, D), lambda i: (i, 0))
    aggp = pl.BlockSpec((NC, _R, D), lambda i: (0, i, 0))
    degp = pl.BlockSpec((NC, _R, 1), lambda i: (0, i, 0))
    w = pl.BlockSpec((D, D), lambda i: (0, 0))
    bias = pl.BlockSpec((1, D), lambda i: (0, 0))
    return rows, aggp, degp, w, bias


def _tc1_call(h0, agg, deg, ws, wn, b):
    rows, aggp, degp, w, bias = _row_specs()
    return pl.pallas_call(
        _tc1_body,
        grid=(N // _R,),
        in_specs=[rows, aggp, degp, w, w, bias],
        out_specs=rows,
        out_shape=jax.ShapeDtypeStruct((N, H), jnp.float32),
    )(h0, agg, deg.reshape(NC, N, 1), ws, wn, b.reshape(1, H))


def _tc2_call(h1, agg, deg, w2s, w2n, b2, wl1, bl1, wl2, bl2):
    rows, aggp, degp, w, bias = _row_specs()
    wc = pl.BlockSpec((H, C), lambda i: (0, 0))
    bc = pl.BlockSpec((1, C), lambda i: (0, 0))
    outc = pl.BlockSpec((_R, C), lambda i: (i, 0))
    return pl.pallas_call(
        _tc2_body,
        grid=(N // _R,),
        in_specs=[rows, aggp, degp, w, w, bias, w, bias, wc, bc],
        out_specs=outc,
        out_shape=jax.ShapeDtypeStruct((N, C), jnp.float32),
    )(h1, agg, deg.reshape(NC, N, 1), w2s, w2n, b2.reshape(1, H),
      wl1, bl1.reshape(1, H), wl2, bl2.reshape(1, C))


def kernel(x, W1_self, W1_neigh, b1, W2_self, W2_neigh, b2,
           W_lin1, b_lin1, W_lin2, b_lin2, n_id, edge_index):
    src = edge_index[0]
    dst = edge_index[1]
    zrows = jnp.zeros((N, D), jnp.float32)
    zdeg = jnp.zeros((N,), jnp.float32)

    h0, deg, agg1 = _make_agg_kernel(True)(x, n_id, src, dst, zrows, zdeg)
    deg = deg.reshape(NC, N)
    h1 = _tc1_call(h0, agg1, deg, W1_self, W1_neigh, b1)
    agg2 = _make_agg_kernel(False)(h1, src, dst, zrows)
    return _tc2_call(h1, agg2, deg, W2_self, W2_neigh, b2,
                     W_lin1, b_lin1, W_lin2, b_lin2)


# speedup vs baseline: 13.5697x; 1.0897x over previous
import functools

import jax
import jax.numpy as jnp
from jax import lax
from jax.experimental import pallas as pl
from jax.experimental.pallas import tpu as pltpu
from jax.experimental.pallas import tpu_sc as plsc

N = 10000
NPAD = 10240
E = 320000
D = 128
H = 128
C = 64

NC = 2
NS = 16
NW = NC * NS
EPT = E // NW
EB = 128
NFULL = EPT // EB
ETAIL = EPT - NFULL * EB
RPS = 624
RTAIL = N - NS * RPS
GB = 128
NGB = N // GB
GTAIL = N - NGB * GB

_MESH = plsc.VectorSubcoreMesh(
    core_axis_name="c", subcore_axis_name="s", num_cores=NC, num_subcores=NS)


def _fill_ones(ref, n):
    for i in range(n // 16):
        ref[pl.ds(i * 16, 16)] = jnp.full((16,), 1.0, jnp.float32)


H0B = 5


def _make_agg_kernel(layer1):
    if layer1:
        out_type = (
            jax.ShapeDtypeStruct((NPAD, D), jnp.float32),
            jax.ShapeDtypeStruct((NC * NPAD,), jnp.float32),
            jax.ShapeDtypeStruct((NC, NPAD, D), jnp.float32),
        )
    else:
        out_type = jax.ShapeDtypeStruct((NC, NPAD, D), jnp.float32)

    scratch = [
        pltpu.VMEM((EPT,), jnp.int32),
        pltpu.VMEM((EB,), jnp.int32),
        pltpu.VMEM((EB,), jnp.int32),
        pltpu.VMEM((ETAIL,), jnp.int32),
        pltpu.VMEM((EB, D), jnp.float32),
        pltpu.VMEM((EB, D), jnp.float32),
        pltpu.VMEM_SHARED((N, D), jnp.float32),
        pltpu.SemaphoreType.DMA,
        pltpu.SemaphoreType.DMA,
        pltpu.SemaphoreType.DMA,
        pltpu.SemaphoreType.DMA,
        pltpu.SemaphoreType.DMA,
        pltpu.SemaphoreType.DMA,
        pltpu.SemaphoreType.DMA,
        pltpu.SemaphoreType.DMA,
    ]
    if layer1:
        scratch += [
            pltpu.VMEM((H0B * GB,), jnp.int32),
            pltpu.VMEM((EB,), jnp.float32),
            pltpu.VMEM((ETAIL,), jnp.float32),
            pltpu.VMEM((RPS,), jnp.float32),
            pltpu.VMEM((RTAIL,), jnp.float32),
            pltpu.VMEM_SHARED((N,), jnp.float32),
            pltpu.SemaphoreType.DMA,
            pltpu.SemaphoreType.DMA,
            pltpu.SemaphoreType.DMA,
        ]

    @functools.partial(pl.kernel, out_type=out_type, mesh=_MESH,
                       scratch_types=scratch)
    def k(*refs):
        if layer1:
            (x_h, n_id_h, edge_h, zrows_h, zdeg_h,
             h0_h, deg_h, agg_h,
             idx_all, dstb0, dstb1, dst_t, rows0, rows1, agg_s,
             s_idx, s_d0, s_d1, s_r0, s_r1, s_s0, s_s1, sem,
             idx_h0, ones_v, ones_t, deg_v, deg_t, deg_s,
             s_h0, s_q0, s_q1) = refs
            table = h0_h
        else:
            (table, edge_h, zrows_h,
             agg_h,
             idx_all, dstb0, dstb1, dst_t, rows0, rows1, agg_s,
             s_idx, s_d0, s_d1, s_r0, s_r1, s_s0, s_s1, sem) = refs

        src_h = edge_h
        dst_h = edge_h.at[pl.ds(E, E)]

        c = lax.axis_index("c")
        s = lax.axis_index("s")
        wid = c * NS + s
        ebase = wid * EPT
        rows = [rows0, rows1]
        dstb = [dstb0, dstb1]
        s_r = [s_r0, s_r1]
        s_s = [s_s0, s_s1]
        s_d = [s_d0, s_d1]

        pltpu.async_copy(src_h.at[pl.ds(ebase, EPT)], idx_all, s_idx)
        pltpu.async_copy(dst_h.at[pl.ds(ebase + NFULL * EB, ETAIL)], dst_t, s_d0)
        if layer1:
            @pl.when(s < NS - 1)
            def _():
                pltpu.async_copy(n_id_h.at[pl.ds(s * H0B * GB, H0B * GB)],
                                 idx_h0, s_h0)

            @pl.when(s == NS - 1)
            def _():
                pltpu.async_copy(
                    n_id_h.at[pl.ds((NS - 1) * H0B * GB, N - (NS - 1) * H0B * GB)],
                    idx_h0.at[pl.ds(0, N - (NS - 1) * H0B * GB)], s_h0)

        pltpu.sync_copy(zrows_h.at[pl.ds(s * RPS, RPS)],
                        agg_s.at[pl.ds(s * RPS, RPS)])

        @pl.when(s == NS - 1)
        def _():
            pltpu.sync_copy(zrows_h.at[pl.ds(NS * RPS, RTAIL)],
                            agg_s.at[pl.ds(NS * RPS, RTAIL)])

        if layer1:
            pltpu.sync_copy(zdeg_h.at[pl.ds(s * RPS, RPS)], deg_v)
            pltpu.sync_copy(deg_v, deg_s.at[pl.ds(s * RPS, RPS)])

            @pl.when(s == NS - 1)
            def _():
                pltpu.sync_copy(zdeg_h.at[pl.ds(NS * RPS, RTAIL)], deg_t)
                pltpu.sync_copy(deg_t, deg_s.at[pl.ds(NS * RPS, RTAIL)])

            _fill_ones(ones_v, EB)
            _fill_ones(ones_t, ETAIL)

            @pl.when(s < NS - 1)
            def _():
                pltpu.make_async_copy(
                    n_id_h.at[pl.ds(s * H0B * GB, H0B * GB)], idx_h0,
                    s_h0).wait()

            @pl.when(s == NS - 1)
            def _():
                pltpu.make_async_copy(
                    n_id_h.at[pl.ds((NS - 1) * H0B * GB, N - (NS - 1) * H0B * GB)],
                    idx_h0.at[pl.ds(0, N - (NS - 1) * H0B * GB)], s_h0).wait()

            for kk in range(H0B):
                b = s * H0B + kk

                @pl.when(b < NGB)
                def _():
                    pltpu.async_copy(
                        x_h.at[idx_h0.at[pl.ds(kk * GB, GB)]],
                        rows[kk % 2], s_r[kk % 2]).wait()
                    pltpu.sync_copy(rows[kk % 2], h0_h.at[pl.ds(b * GB, GB)])

            @pl.when(s == NS - 1)
            def _():
                pltpu.async_copy(
                    x_h.at[idx_h0.at[pl.ds((NGB - (NS - 1) * H0B) * GB, GTAIL)]],
                    rows0.at[pl.ds(0, GTAIL), :], sem).wait()
                pltpu.sync_copy(rows0.at[pl.ds(0, GTAIL), :],
                                h0_h.at[pl.ds(NGB * GB, GTAIL)])

        plsc.subcore_barrier()

        pltpu.make_async_copy(src_h.at[pl.ds(ebase, EPT)], idx_all,
                              s_idx).wait()

        def fire_d(i, b):
            pltpu.async_copy(dst_h.at[pl.ds(ebase + i * EB, EB)],
                             dstb[b], s_d[b])

        def wait_d(i, b):
            pltpu.make_async_copy(dst_h.at[pl.ds(ebase + i * EB, EB)],
                                  dstb[b], s_d[b]).wait()

        def fire_g(i, b):
            pltpu.async_copy(table.at[idx_all.at[pl.ds(i * EB, EB)]],
                             rows[b], s_r[b])

        def wait_g(i, b):
            pltpu.make_async_copy(table.at[idx_all.at[pl.ds(i * EB, EB)]],
                                  rows[b], s_r[b]).wait()

        def fire_s(i, b):
            pltpu.async_copy(rows[b], agg_s.at[dstb[b]], s_s[b], add=True)
            if layer1:
                pltpu.async_copy(ones_v, deg_s.at[dstb[b]],
                                 [s_q0, s_q1][b], add=True)

        def wait_s(i, b):
            pltpu.make_async_copy(rows[b], agg_s.at[dstb[b]], s_s[b]).wait()
            if layer1:
                pltpu.make_async_copy(ones_v, deg_s.at[dstb[b]],
                                      [s_q0, s_q1][b]).wait()

        def fire_batch(i, b):
            fire_d(i, b)
            fire_g(i, b)

        fire_batch(0, 0)

        def grp(g, carry):
            i = 2 * g
            pl.when(g >= 1)(lambda: wait_s(i - 1, 1))
            fire_batch(i + 1, 1)
            wait_g(i, 0)
            wait_d(i, 0)
            fire_s(i, 0)
            i = 2 * g + 1
            wait_s(i - 1, 0)
            pl.when(g < NFULL // 2 - 1)(lambda: fire_batch(i + 1, 0))
            wait_g(i, 1)
            wait_d(i, 1)
            fire_s(i, 1)
            return carry

        lax.fori_loop(0, NFULL // 2, grp, 0)
        wait_s(NFULL - 1, 1)

        pltpu.make_async_copy(dst_h.at[pl.ds(ebase + NFULL * EB, ETAIL)],
                              dst_t, s_d0).wait()
        pltpu.async_copy(table.at[idx_all.at[pl.ds(NFULL * EB, ETAIL)]],
                         rows0.at[pl.ds(0, ETAIL), :], sem).wait()
        pltpu.sync_copy(rows0.at[pl.ds(0, ETAIL), :], agg_s.at[dst_t], add=True)
        if layer1:
            pltpu.sync_copy(ones_t, deg_s.at[dst_t], add=True)

        plsc.subcore_barrier()

        pltpu.sync_copy(agg_s.at[pl.ds(s * RPS, RPS)],
                        agg_h.at[c, pl.ds(s * RPS, RPS)])

        @pl.when(s == NS - 1)
        def _():
            pltpu.sync_copy(agg_s.at[pl.ds(NS * RPS, RTAIL)],
                            agg_h.at[c, pl.ds(NS * RPS, RTAIL)])

        if layer1:
            pltpu.sync_copy(deg_s.at[pl.ds(s * RPS, RPS)], deg_v)
            pltpu.sync_copy(deg_v, deg_h.at[pl.ds(c * NPAD + s * RPS, RPS)])

            @pl.when(s == NS - 1)
            def _():
                pltpu.sync_copy(deg_s.at[pl.ds(NS * RPS, RTAIL)], deg_t)
                pltpu.sync_copy(deg_t,
                                deg_h.at[pl.ds(c * NPAD + NS * RPS, RTAIL)])

    return k


_R = 1280


def _tc1_body(h0, aggp, degp, ws, wn, b, out):
    agg = aggp[0] + aggp[1]
    degs = degp[...]
    deg = (degs[0] + degs[1])[:, None]
    mean = agg / jnp.maximum(deg, 1.0)
    acc = jnp.dot(h0[...], ws[...], preferred_element_type=jnp.float32)
    acc += jnp.dot(mean, wn[...], preferred_element_type=jnp.float32)
    out[...] = jnp.maximum(acc + b[...], 0.0)


def _tc2_body(h1, aggp, degp, w2s, w2n, b2, wl1, bl1, wl2, bl2, out):
    agg = aggp[0] + aggp[1]
    degs = degp[...]
    deg = (degs[0] + degs[1])[:, None]
    mean = agg / jnp.maximum(deg, 1.0)
    h2 = jnp.dot(h1[...], w2s[...], preferred_element_type=jnp.float32)
    h2 += jnp.dot(mean, w2n[...], preferred_element_type=jnp.float32)
    h2 = jnp.maximum(h2 + b2[...], 0.0)
    h3 = jnp.maximum(
        jnp.dot(h2, wl1[...], preferred_element_type=jnp.float32) + bl1[...], 0.0)
    logits = jnp.dot(h3, wl2[...], preferred_element_type=jnp.float32) + bl2[...]
    m = jnp.max(logits, axis=-1, keepdims=True)
    lse = jnp.log(jnp.sum(jnp.exp(logits - m), axis=-1, keepdims=True)) + m
    out[...] = logits - lse


def _row_specs():
    rows = pl.BlockSpec((_R, D), lambda i: (i, 0))
    aggp = pl.BlockSpec((NC, _R, D), lambda i: (0, i, 0))
    degp = pl.BlockSpec((NC, _R), lambda i: (0, i))
    w = pl.BlockSpec((D, D), lambda i: (0, 0))
    bias = pl.BlockSpec((1, D), lambda i: (0, 0))
    return rows, aggp, degp, w, bias


def _tc1_call(h0, agg, deg, ws, wn, b):
    rows, aggp, degp, w, bias = _row_specs()
    return pl.pallas_call(
        _tc1_body,
        grid=(NPAD // _R,),
        in_specs=[rows, aggp, degp, w, w, bias],
        out_specs=rows,
        out_shape=jax.ShapeDtypeStruct((NPAD, H), jnp.float32),
    )(h0, agg, deg, ws, wn, b.reshape(1, H))


def _tc2_call(h1, agg, deg, w2s, w2n, b2, wl1, bl1, wl2, bl2):
    rows, aggp, degp, w, bias = _row_specs()
    wc = pl.BlockSpec((H, C), lambda i: (0, 0))
    bc = pl.BlockSpec((1, C), lambda i: (0, 0))
    outc = pl.BlockSpec((_R, C), lambda i: (i, 0))
    return pl.pallas_call(
        _tc2_body,
        grid=(NPAD // _R,),
        in_specs=[rows, aggp, degp, w, w, bias, w, bias, wc, bc],
        out_specs=outc,
        out_shape=jax.ShapeDtypeStruct((NPAD, C), jnp.float32),
    )(h1, agg, deg, w2s, w2n, b2.reshape(1, H),
      wl1, bl1.reshape(1, H), wl2, bl2.reshape(1, C))


def kernel(x, W1_self, W1_neigh, b1, W2_self, W2_neigh, b2,
           W_lin1, b_lin1, W_lin2, b_lin2, n_id, edge_index):
    edge_flat = edge_index.reshape(2 * E)
    zrows = jnp.zeros((N, D), jnp.float32)
    zdeg = jnp.zeros((N,), jnp.float32)

    h0, deg, agg1 = _make_agg_kernel(True)(x, n_id, edge_flat, zrows, zdeg)
    deg = deg.reshape(NC, NPAD)
    h1 = _tc1_call(h0, agg1, deg, W1_self, W1_neigh, b1)
    agg2 = _make_agg_kernel(False)(h1, edge_flat, zrows)
    out = _tc2_call(h1, agg2, deg, W2_self, W2_neigh, b2,
                    W_lin1, b_lin1, W_lin2, b_lin2)
    return out[:N]


# speedup vs baseline: 13.7996x; 1.0169x over previous
import functools

import jax
import jax.numpy as jnp
from jax import lax
from jax.experimental import pallas as pl
from jax.experimental.pallas import tpu as pltpu
from jax.experimental.pallas import tpu_sc as plsc

N = 10000
NPAD = 10240
E = 320000
D = 128
H = 128
C = 64

NC = 2
NS = 16
NW = NC * NS
EB = 128
NB = E // EB
NFULL = NB // NW
NBMAX = NFULL + 1
RPS = 624
RTAIL = N - NS * RPS
GB = 128
NGB = N // GB
GTAIL = N - NGB * GB

_MESH = plsc.VectorSubcoreMesh(
    core_axis_name="c", subcore_axis_name="s", num_cores=NC, num_subcores=NS)


def _fill_ones(ref, n):
    for i in range(n // 16):
        ref[pl.ds(i * 16, 16)] = jnp.full((16,), 1.0, jnp.float32)


H0B = 5


def _make_agg_kernel(layer1):
    if layer1:
        out_type = (
            jax.ShapeDtypeStruct((NPAD, D), jnp.float32),
            jax.ShapeDtypeStruct((NC * NPAD,), jnp.float32),
            jax.ShapeDtypeStruct((NC, NPAD, D), jnp.float32),
        )
    else:
        out_type = jax.ShapeDtypeStruct((NC, NPAD, D), jnp.float32)

    scratch = [
        pltpu.VMEM((NBMAX * EB,), jnp.int32),
        pltpu.VMEM((2, EB), jnp.int32),
        pltpu.VMEM((2, EB), jnp.int32),
        pltpu.VMEM((EB, D), jnp.float32),
        pltpu.VMEM((EB, D), jnp.float32),
        pltpu.VMEM_SHARED((N, D), jnp.float32),
        pltpu.SemaphoreType.DMA,
        pltpu.SemaphoreType.DMA,
        pltpu.SemaphoreType.DMA,
        pltpu.SemaphoreType.DMA,
        pltpu.SemaphoreType.DMA,
        pltpu.SemaphoreType.DMA,
        pltpu.SemaphoreType.DMA,
        pltpu.SemaphoreType.DMA,
    ]
    if layer1:
        scratch += [
            pltpu.VMEM((H0B * GB,), jnp.int32),
            pltpu.VMEM((EB,), jnp.float32),
            pltpu.VMEM((RPS,), jnp.float32),
            pltpu.VMEM((RTAIL,), jnp.float32),
            pltpu.VMEM_SHARED((N,), jnp.float32),
            pltpu.SemaphoreType.DMA,
            pltpu.SemaphoreType.DMA,
            pltpu.SemaphoreType.DMA,
        ]

    @functools.partial(pl.kernel, out_type=out_type, mesh=_MESH,
                       scratch_types=scratch)
    def k(*refs):
        if layer1:
            (x_h, n_id_h, edge_h, zrows_h, zdeg_h,
             h0_h, deg_h, agg_h,
             idx_all, edgeb0, edgeb1, rows0, rows1, agg_s,
             s_idx, s_d0, s_d1, s_r0, s_r1, s_s0, s_s1, sem,
             idx_h0, ones_v, deg_v, deg_t, deg_s,
             s_h0, s_q0, s_q1) = refs
            table = h0_h
        else:
            (table, edge_h, zrows_h,
             agg_h,
             idx_all, edgeb0, edgeb1, rows0, rows1, agg_s,
             s_idx, s_d0, s_d1, s_r0, s_r1, s_s0, s_s1, sem) = refs

        c = lax.axis_index("c")
        s = lax.axis_index("s")
        wid = c * NS + s
        has_extra = (wid % 8) == 0
        sb = NFULL * wid + ((wid >= 1).astype(jnp.int32)
                            + (wid >= 9).astype(jnp.int32)
                            + (wid >= 17).astype(jnp.int32)
                            + (wid >= 25).astype(jnp.int32))
        rows = [rows0, rows1]
        edgeb = [edgeb0, edgeb1]
        s_r = [s_r0, s_r1]
        s_s = [s_s0, s_s1]
        s_d = [s_d0, s_d1]

        pltpu.async_copy(edge_h.at[0, pl.ds(sb * EB, NFULL * EB)],
                         idx_all.at[pl.ds(0, NFULL * EB)], s_idx)

        @pl.when(has_extra)
        def _():
            pltpu.async_copy(edge_h.at[0, pl.ds((sb + NFULL) * EB, EB)],
                             idx_all.at[pl.ds(NFULL * EB, EB)], s_idx)

        if layer1:
            @pl.when(s < NS - 1)
            def _():
                pltpu.async_copy(n_id_h.at[pl.ds(s * H0B * GB, H0B * GB)],
                                 idx_h0, s_h0)

            @pl.when(s == NS - 1)
            def _():
                pltpu.async_copy(
                    n_id_h.at[pl.ds((NS - 1) * H0B * GB, N - (NS - 1) * H0B * GB)],
                    idx_h0.at[pl.ds(0, N - (NS - 1) * H0B * GB)], s_h0)

        pltpu.sync_copy(zrows_h, agg_s.at[pl.ds(s * RPS, RPS)])

        @pl.when(s == NS - 1)
        def _():
            pltpu.sync_copy(zrows_h.at[pl.ds(0, RTAIL)],
                            agg_s.at[pl.ds(NS * RPS, RTAIL)])

        if layer1:
            pltpu.sync_copy(zdeg_h, deg_v)
            pltpu.sync_copy(deg_v, deg_s.at[pl.ds(s * RPS, RPS)])

            @pl.when(s == NS - 1)
            def _():
                pltpu.sync_copy(zdeg_h.at[pl.ds(0, RTAIL)], deg_t)
                pltpu.sync_copy(deg_t, deg_s.at[pl.ds(NS * RPS, RTAIL)])

            _fill_ones(ones_v, EB)

            @pl.when(s < NS - 1)
            def _():
                pltpu.make_async_copy(
                    n_id_h.at[pl.ds(s * H0B * GB, H0B * GB)], idx_h0,
                    s_h0).wait()

            @pl.when(s == NS - 1)
            def _():
                pltpu.make_async_copy(
                    n_id_h.at[pl.ds((NS - 1) * H0B * GB, N - (NS - 1) * H0B * GB)],
                    idx_h0.at[pl.ds(0, N - (NS - 1) * H0B * GB)], s_h0).wait()

            for kk in range(H0B):
                b = s * H0B + kk

                @pl.when(b < NGB)
                def _():
                    pltpu.async_copy(
                        x_h.at[idx_h0.at[pl.ds(kk * GB, GB)]],
                        rows[kk % 2], s_r[kk % 2]).wait()
                    pltpu.sync_copy(rows[kk % 2], h0_h.at[pl.ds(b * GB, GB)])

            @pl.when(s == NS - 1)
            def _():
                pltpu.async_copy(
                    x_h.at[idx_h0.at[pl.ds((NGB - (NS - 1) * H0B) * GB, GTAIL)]],
                    rows0.at[pl.ds(0, GTAIL), :], sem).wait()
                pltpu.sync_copy(rows0.at[pl.ds(0, GTAIL), :],
                                h0_h.at[pl.ds(NGB * GB, GTAIL)])

        plsc.subcore_barrier()

        pltpu.make_async_copy(edge_h.at[0, pl.ds(sb * EB, NFULL * EB)],
                              idx_all.at[pl.ds(0, NFULL * EB)], s_idx).wait()

        @pl.when(has_extra)
        def _():
            pltpu.make_async_copy(edge_h.at[0, pl.ds((sb + NFULL) * EB, EB)],
                                  idx_all.at[pl.ds(NFULL * EB, EB)],
                                  s_idx).wait()

        def fire_d(i, b):
            pltpu.async_copy(edge_h.at[:, pl.ds((sb + i) * EB, EB)],
                             edgeb[b], s_d[b])

        def wait_d(i, b):
            pltpu.make_async_copy(edge_h.at[:, pl.ds((sb + i) * EB, EB)],
                                  edgeb[b], s_d[b]).wait()

        def fire_g(i, b):
            pltpu.async_copy(table.at[idx_all.at[pl.ds(i * EB, EB)]],
                             rows[b], s_r[b])

        def wait_g(i, b):
            pltpu.make_async_copy(table.at[idx_all.at[pl.ds(i * EB, EB)]],
                                  rows[b], s_r[b]).wait()

        def fire_s(i, b):
            pltpu.async_copy(rows[b], agg_s.at[edgeb[b].at[1]], s_s[b],
                             add=True)
            if layer1:
                pltpu.async_copy(ones_v, deg_s.at[edgeb[b].at[1]],
                                 [s_q0, s_q1][b], add=True)

        def wait_s(i, b):
            pltpu.make_async_copy(rows[b], agg_s.at[edgeb[b].at[1]],
                                  s_s[b]).wait()
            if layer1:
                pltpu.make_async_copy(ones_v, deg_s.at[edgeb[b].at[1]],
                                      [s_q0, s_q1][b]).wait()

        def fire_batch(i, b):
            fire_d(i, b)
            fire_g(i, b)

        fire_batch(0, 0)

        def grp(g, carry):
            i = 2 * g
            pl.when(g >= 1)(lambda: wait_s(i - 1, 1))
            fire_batch(i + 1, 1)
            wait_g(i, 0)
            wait_d(i, 0)
            fire_s(i, 0)
            i = 2 * g + 1
            wait_s(i - 1, 0)
            pl.when(g < NFULL // 2 - 1)(lambda: fire_batch(i + 1, 0))
            wait_g(i, 1)
            wait_d(i, 1)
            fire_s(i, 1)
            return carry

        lax.fori_loop(0, NFULL // 2, grp, 0)
        wait_s(NFULL - 1, 1)

        @pl.when(has_extra)
        def _():
            fire_batch(NFULL, 0)
            wait_g(NFULL, 0)
            wait_d(NFULL, 0)
            fire_s(NFULL, 0)
            wait_s(NFULL, 0)

        plsc.subcore_barrier()

        pltpu.sync_copy(agg_s.at[pl.ds(s * RPS, RPS)],
                        agg_h.at[c, pl.ds(s * RPS, RPS)])

        @pl.when(s == NS - 1)
        def _():
            pltpu.sync_copy(agg_s.at[pl.ds(NS * RPS, RTAIL)],
                            agg_h.at[c, pl.ds(NS * RPS, RTAIL)])

        if layer1:
            pltpu.sync_copy(deg_s.at[pl.ds(s * RPS, RPS)], deg_v)
            pltpu.sync_copy(deg_v, deg_h.at[pl.ds(c * NPAD + s * RPS, RPS)])

            @pl.when(s == NS - 1)
            def _():
                pltpu.sync_copy(deg_s.at[pl.ds(NS * RPS, RTAIL)], deg_t)
                pltpu.sync_copy(deg_t,
                                deg_h.at[pl.ds(c * NPAD + NS * RPS, RTAIL)])

    return k


_R = 1280


def _deg_block(degp):
    i = pl.program_id(0)
    d0 = degp[pl.ds(i * _R, _R)]
    d1 = degp[pl.ds(NPAD + i * _R, _R)]
    return (d0 + d1)[:, None]


def _tc1_body(h0, aggp, degp, ws, wn, b, out):
    agg = aggp[0] + aggp[1]
    deg = _deg_block(degp)
    mean = agg / jnp.maximum(deg, 1.0)
    acc = jnp.dot(h0[...], ws[...], preferred_element_type=jnp.float32)
    acc += jnp.dot(mean, wn[...], preferred_element_type=jnp.float32)
    out[...] = jnp.maximum(acc + b[...], 0.0)


def _tc2_body(h1, aggp, degp, w2s, w2n, b2, wl1, bl1, wl2, bl2, out):
    agg = aggp[0] + aggp[1]
    deg = _deg_block(degp)
    mean = agg / jnp.maximum(deg, 1.0)
    h2 = jnp.dot(h1[...], w2s[...], preferred_element_type=jnp.float32)
    h2 += jnp.dot(mean, w2n[...], preferred_element_type=jnp.float32)
    h2 = jnp.maximum(h2 + b2[...], 0.0)
    h3 = jnp.maximum(
        jnp.dot(h2, wl1[...], preferred_element_type=jnp.float32) + bl1[...], 0.0)
    logits = jnp.dot(h3, wl2[...], preferred_element_type=jnp.float32) + bl2[...]
    m = jnp.max(logits, axis=-1, keepdims=True)
    lse = jnp.log(jnp.sum(jnp.exp(logits - m), axis=-1, keepdims=True)) + m
    out[...] = logits - lse


def _row_specs():
    rows = pl.BlockSpec((_R, D), lambda i: (i, 0))
    aggp = pl.BlockSpec((NC, _R, D), lambda i: (0, i, 0))
    degp = pl.BlockSpec((NC * NPAD,), lambda i: (0,))
    w = pl.BlockSpec((D, D), lambda i: (0, 0))
    bias = pl.BlockSpec((1, D), lambda i: (0, 0))
    return rows, aggp, degp, w, bias


def _tc1_call(h0, agg, deg, ws, wn, b):
    rows, aggp, degp, w, bias = _row_specs()
    return pl.pallas_call(
        _tc1_body,
        grid=(NPAD // _R,),
        in_specs=[rows, aggp, degp, w, w, bias],
        out_specs=rows,
        out_shape=jax.ShapeDtypeStruct((NPAD, H), jnp.float32),
    )(h0, agg, deg, ws, wn, b.reshape(1, H))


def _tc2_call(h1, agg, deg, w2s, w2n, b2, wl1, bl1, wl2, bl2):
    rows, aggp, degp, w, bias = _row_specs()
    wc = pl.BlockSpec((H, C), lambda i: (0, 0))
    bc = pl.BlockSpec((1, C), lambda i: (0, 0))
    outc = pl.BlockSpec((_R, C), lambda i: (i, 0))
    return pl.pallas_call(
        _tc2_body,
        grid=(NPAD // _R,),
        in_specs=[rows, aggp, degp, w, w, bias, w, bias, wc, bc],
        out_specs=outc,
        out_shape=jax.ShapeDtypeStruct((N, C), jnp.float32),
    )(h1, agg, deg, w2s, w2n, b2.reshape(1, H),
      wl1, bl1.reshape(1, H), wl2, bl2.reshape(1, C))


def kernel(x, W1_self, W1_neigh, b1, W2_self, W2_neigh, b2,
           W_lin1, b_lin1, W_lin2, b_lin2, n_id, edge_index):
    zrows = jnp.zeros((RPS, D), jnp.float32)
    zdeg = jnp.zeros((RPS,), jnp.float32)

    h0, deg, agg1 = _make_agg_kernel(True)(x, n_id, edge_index, zrows, zdeg)
    h1 = _tc1_call(h0, agg1, deg, W1_self, W1_neigh, b1)
    agg2 = _make_agg_kernel(False)(h1, edge_index, zrows)
    return _tc2_call(h1, agg2, deg, W2_self, W2_neigh, b2,
                     W_lin1, b_lin1, W_lin2, b_lin2)


# speedup vs baseline: 13.9126x; 1.0082x over previous
import functools

import jax
import jax.numpy as jnp
from jax import lax
from jax.experimental import pallas as pl
from jax.experimental.pallas import tpu as pltpu
from jax.experimental.pallas import tpu_sc as plsc

N = 10000
NPAD = 10240
E = 320000
D = 128
H = 128
C = 64

NC = 2
NS = 16
NW = NC * NS
EB = 128
NB = E // EB
NFULL = NB // NW
NBMAX = NFULL + 1
RPS = 624
RTAIL = N - NS * RPS
GB = 128
NGB = N // GB
GTAIL = N - NGB * GB

_MESH = plsc.VectorSubcoreMesh(
    core_axis_name="c", subcore_axis_name="s", num_cores=NC, num_subcores=NS)


def _fill_ones(ref, n):
    for i in range(n // 16):
        ref[pl.ds(i * 16, 16)] = jnp.full((16,), 1.0, jnp.float32)


H0B = 5


def _make_agg_kernel(layer1):
    if layer1:
        out_type = (
            jax.ShapeDtypeStruct((NPAD, D), jnp.float32),
            jax.ShapeDtypeStruct((NC * NPAD,), jnp.float32),
            jax.ShapeDtypeStruct((NC, NPAD, D), jnp.float32),
        )
    else:
        out_type = jax.ShapeDtypeStruct((NC, NPAD, D), jnp.float32)

    scratch = [
        pltpu.VMEM((NBMAX * EB,), jnp.int32),
        pltpu.VMEM((2, EB), jnp.int32),
        pltpu.VMEM((2, EB), jnp.int32),
        pltpu.VMEM((EB, D), jnp.float32),
        pltpu.VMEM((EB, D), jnp.float32),
        pltpu.VMEM_SHARED((N, D), jnp.float32),
        pltpu.SemaphoreType.DMA,
        pltpu.SemaphoreType.DMA,
        pltpu.SemaphoreType.DMA,
        pltpu.SemaphoreType.DMA,
        pltpu.SemaphoreType.DMA,
        pltpu.SemaphoreType.DMA,
        pltpu.SemaphoreType.DMA,
        pltpu.SemaphoreType.DMA,
    ]
    if layer1:
        scratch += [
            pltpu.VMEM((H0B * GB,), jnp.int32),
            pltpu.VMEM((EB,), jnp.float32),
            pltpu.VMEM((RPS,), jnp.float32),
            pltpu.VMEM((RTAIL,), jnp.float32),
            pltpu.VMEM_SHARED((N,), jnp.float32),
            pltpu.SemaphoreType.DMA,
            pltpu.SemaphoreType.DMA,
            pltpu.SemaphoreType.DMA,
        ]

    @functools.partial(pl.kernel, out_type=out_type, mesh=_MESH,
                       scratch_types=scratch)
    def k(*refs):
        if layer1:
            (x_h, n_id_h, edge_h, zrows_h, zdeg_h,
             h0_h, deg_h, agg_h,
             idx_all, edgeb0, edgeb1, rows0, rows1, agg_s,
             s_idx, s_d0, s_d1, s_r0, s_r1, s_s0, s_s1, sem,
             idx_h0, ones_v, deg_v, deg_t, deg_s,
             s_h0, s_q0, s_q1) = refs
            table = h0_h
        else:
            (table, edge_h, zrows_h,
             agg_h,
             idx_all, edgeb0, edgeb1, rows0, rows1, agg_s,
             s_idx, s_d0, s_d1, s_r0, s_r1, s_s0, s_s1, sem) = refs

        c = lax.axis_index("c")
        s = lax.axis_index("s")
        wid = c * NS + s
        has_extra = (wid % 8) == 0
        sb = NFULL * wid + ((wid >= 1).astype(jnp.int32)
                            + (wid >= 9).astype(jnp.int32)
                            + (wid >= 17).astype(jnp.int32)
                            + (wid >= 25).astype(jnp.int32))
        rows = [rows0, rows1]
        edgeb = [edgeb0, edgeb1]
        s_r = [s_r0, s_r1]
        s_s = [s_s0, s_s1]
        s_d = [s_d0, s_d1]

        pltpu.async_copy(edge_h.at[0, pl.ds(sb * EB, NFULL * EB)],
                         idx_all.at[pl.ds(0, NFULL * EB)], s_idx)

        @pl.when(has_extra)
        def _():
            pltpu.async_copy(edge_h.at[0, pl.ds((sb + NFULL) * EB, EB)],
                             idx_all.at[pl.ds(NFULL * EB, EB)], s_idx)

        if layer1:
            @pl.when(s < NS - 1)
            def _():
                pltpu.async_copy(n_id_h.at[pl.ds(s * H0B * GB, H0B * GB)],
                                 idx_h0, s_h0)

            @pl.when(s == NS - 1)
            def _():
                pltpu.async_copy(
                    n_id_h.at[pl.ds((NS - 1) * H0B * GB, N - (NS - 1) * H0B * GB)],
                    idx_h0.at[pl.ds(0, N - (NS - 1) * H0B * GB)], s_h0)

        pltpu.sync_copy(zrows_h, agg_s.at[pl.ds(s * RPS, RPS)])

        @pl.when(s == NS - 1)
        def _():
            pltpu.sync_copy(zrows_h.at[pl.ds(0, RTAIL)],
                            agg_s.at[pl.ds(NS * RPS, RTAIL)])

        if layer1:
            pltpu.sync_copy(zdeg_h, deg_v)
            pltpu.sync_copy(deg_v, deg_s.at[pl.ds(s * RPS, RPS)])

            @pl.when(s == NS - 1)
            def _():
                pltpu.sync_copy(zdeg_h.at[pl.ds(0, RTAIL)], deg_t)
                pltpu.sync_copy(deg_t, deg_s.at[pl.ds(NS * RPS, RTAIL)])

            _fill_ones(ones_v, EB)

            @pl.when(s < NS - 1)
            def _():
                pltpu.make_async_copy(
                    n_id_h.at[pl.ds(s * H0B * GB, H0B * GB)], idx_h0,
                    s_h0).wait()

            @pl.when(s == NS - 1)
            def _():
                pltpu.make_async_copy(
                    n_id_h.at[pl.ds((NS - 1) * H0B * GB, N - (NS - 1) * H0B * GB)],
                    idx_h0.at[pl.ds(0, N - (NS - 1) * H0B * GB)], s_h0).wait()

            def h0_g(kk):
                return pltpu.make_async_copy(
                    x_h.at[idx_h0.at[pl.ds(kk * GB, GB)]],
                    rows[kk % 2], s_r[kk % 2])

            def h0_w(kk):
                return pltpu.make_async_copy(
                    rows[kk % 2],
                    h0_h.at[pl.ds((s * H0B + kk) * GB, GB)], s_s[kk % 2])

            pltpu.async_copy(x_h.at[idx_h0.at[pl.ds(0, GB)]], rows0, s_r0)
            for kk in range(H0B):
                b = s * H0B + kk

                @pl.when(b < NGB)
                def _():
                    h0_g(kk).wait()
                    if kk >= 1:
                        h0_w(kk - 1).wait()
                    if kk + 1 < H0B:
                        pl.when(b + 1 < NGB)(lambda: h0_g(kk + 1).start())
                    h0_w(kk).start()

            pl.when(s < NS - 1)(lambda: h0_w(H0B - 1).wait())
            pl.when(s == NS - 1)(lambda: h0_w(NGB - (NS - 1) * H0B - 1).wait())

            @pl.when(s == NS - 1)
            def _():
                pltpu.async_copy(
                    x_h.at[idx_h0.at[pl.ds((NGB - (NS - 1) * H0B) * GB, GTAIL)]],
                    rows0.at[pl.ds(0, GTAIL), :], sem).wait()
                pltpu.sync_copy(rows0.at[pl.ds(0, GTAIL), :],
                                h0_h.at[pl.ds(NGB * GB, GTAIL)])

        plsc.subcore_barrier()

        pltpu.make_async_copy(edge_h.at[0, pl.ds(sb * EB, NFULL * EB)],
                              idx_all.at[pl.ds(0, NFULL * EB)], s_idx).wait()

        @pl.when(has_extra)
        def _():
            pltpu.make_async_copy(edge_h.at[0, pl.ds((sb + NFULL) * EB, EB)],
                                  idx_all.at[pl.ds(NFULL * EB, EB)],
                                  s_idx).wait()

        def fire_d(i, b):
            pltpu.async_copy(edge_h.at[:, pl.ds((sb + i) * EB, EB)],
                             edgeb[b], s_d[b])

        def wait_d(i, b):
            pltpu.make_async_copy(edge_h.at[:, pl.ds((sb + i) * EB, EB)],
                                  edgeb[b], s_d[b]).wait()

        def fire_g(i, b):
            pltpu.async_copy(table.at[idx_all.at[pl.ds(i * EB, EB)]],
                             rows[b], s_r[b])

        def wait_g(i, b):
            pltpu.make_async_copy(table.at[idx_all.at[pl.ds(i * EB, EB)]],
                                  rows[b], s_r[b]).wait()

        def fire_s(i, b):
            pltpu.async_copy(rows[b], agg_s.at[edgeb[b].at[1]], s_s[b],
                             add=True)
            if layer1:
                pltpu.async_copy(ones_v, deg_s.at[edgeb[b].at[1]],
                                 [s_q0, s_q1][b], add=True)

        def wait_s(i, b):
            pltpu.make_async_copy(rows[b], agg_s.at[edgeb[b].at[1]],
                                  s_s[b]).wait()
            if layer1:
                pltpu.make_async_copy(ones_v, deg_s.at[edgeb[b].at[1]],
                                      [s_q0, s_q1][b]).wait()

        def fire_batch(i, b):
            fire_d(i, b)
            fire_g(i, b)

        fire_batch(0, 0)

        def grp(g, carry):
            i = 2 * g
            pl.when(g >= 1)(lambda: wait_s(i - 1, 1))
            fire_batch(i + 1, 1)
            wait_g(i, 0)
            wait_d(i, 0)
            fire_s(i, 0)
            i = 2 * g + 1
            wait_s(i - 1, 0)
            pl.when(g < NFULL // 2 - 1)(lambda: fire_batch(i + 1, 0))
            wait_g(i, 1)
            wait_d(i, 1)
            fire_s(i, 1)
            return carry

        lax.fori_loop(0, NFULL // 2, grp, 0)
        wait_s(NFULL - 1, 1)

        @pl.when(has_extra)
        def _():
            fire_batch(NFULL, 0)
            wait_g(NFULL, 0)
            wait_d(NFULL, 0)
            fire_s(NFULL, 0)
            wait_s(NFULL, 0)

        plsc.subcore_barrier()

        pltpu.sync_copy(agg_s.at[pl.ds(s * RPS, RPS)],
                        agg_h.at[c, pl.ds(s * RPS, RPS)])

        @pl.when(s == NS - 1)
        def _():
            pltpu.sync_copy(agg_s.at[pl.ds(NS * RPS, RTAIL)],
                            agg_h.at[c, pl.ds(NS * RPS, RTAIL)])

        if layer1:
            pltpu.sync_copy(deg_s.at[pl.ds(s * RPS, RPS)], deg_v)
            pltpu.sync_copy(deg_v, deg_h.at[pl.ds(c * NPAD + s * RPS, RPS)])

            @pl.when(s == NS - 1)
            def _():
                pltpu.sync_copy(deg_s.at[pl.ds(NS * RPS, RTAIL)], deg_t)
                pltpu.sync_copy(deg_t,
                                deg_h.at[pl.ds(c * NPAD + NS * RPS, RTAIL)])

    return k


_R = 1280


def _deg_block(degp):
    i = pl.program_id(0)
    d0 = degp[pl.ds(i * _R, _R)]
    d1 = degp[pl.ds(NPAD + i * _R, _R)]
    return (d0 + d1)[:, None]


def _tc1_body(h0, aggp, degp, ws, wn, b, out):
    agg = aggp[0] + aggp[1]
    deg = _deg_block(degp)
    mean = agg / jnp.maximum(deg, 1.0)
    acc = jnp.dot(h0[...], ws[...], preferred_element_type=jnp.float32)
    acc += jnp.dot(mean, wn[...], preferred_element_type=jnp.float32)
    out[...] = jnp.maximum(acc + b[...], 0.0)


def _tc2_body(h1, aggp, degp, w2s, w2n, b2, wl1, bl1, wl2, bl2, out):
    agg = aggp[0] + aggp[1]
    deg = _deg_block(degp)
    mean = agg / jnp.maximum(deg, 1.0)
    h2 = jnp.dot(h1[...], w2s[...], preferred_element_type=jnp.float32)
    h2 += jnp.dot(mean, w2n[...], preferred_element_type=jnp.float32)
    h2 = jnp.maximum(h2 + b2[...], 0.0)
    h3 = jnp.maximum(
        jnp.dot(h2, wl1[...], preferred_element_type=jnp.float32) + bl1[...], 0.0)
    logits = jnp.dot(h3, wl2[...], preferred_element_type=jnp.float32) + bl2[...]
    m = jnp.max(logits, axis=-1, keepdims=True)
    lse = jnp.log(jnp.sum(jnp.exp(logits - m), axis=-1, keepdims=True)) + m
    out[...] = logits - lse


def _row_specs():
    rows = pl.BlockSpec((_R, D), lambda i: (i, 0))
    aggp = pl.BlockSpec((NC, _R, D), lambda i: (0, i, 0))
    degp = pl.BlockSpec((NC * NPAD,), lambda i: (0,))
    w = pl.BlockSpec((D, D), lambda i: (0, 0))
    bias = pl.BlockSpec((1, D), lambda i: (0, 0))
    return rows, aggp, degp, w, bias


def _tc1_call(h0, agg, deg, ws, wn, b):
    rows, aggp, degp, w, bias = _row_specs()
    return pl.pallas_call(
        _tc1_body,
        grid=(NPAD // _R,),
        in_specs=[rows, aggp, degp, w, w, bias],
        out_specs=rows,
        out_shape=jax.ShapeDtypeStruct((NPAD, H), jnp.float32),
    )(h0, agg, deg, ws, wn, b.reshape(1, H))


def _tc2_call(h1, agg, deg, w2s, w2n, b2, wl1, bl1, wl2, bl2):
    rows, aggp, degp, w, bias = _row_specs()
    wc = pl.BlockSpec((H, C), lambda i: (0, 0))
    bc = pl.BlockSpec((1, C), lambda i: (0, 0))
    outc = pl.BlockSpec((_R, C), lambda i: (i, 0))
    return pl.pallas_call(
        _tc2_body,
        grid=(NPAD // _R,),
        in_specs=[rows, aggp, degp, w, w, bias, w, bias, wc, bc],
        out_specs=outc,
        out_shape=jax.ShapeDtypeStruct((N, C), jnp.float32),
    )(h1, agg, deg, w2s, w2n, b2.reshape(1, H),
      wl1, bl1.reshape(1, H), wl2, bl2.reshape(1, C))


def kernel(x, W1_self, W1_neigh, b1, W2_self, W2_neigh, b2,
           W_lin1, b_lin1, W_lin2, b_lin2, n_id, edge_index):
    zrows = jnp.zeros((RPS, D), jnp.float32)
    zdeg = jnp.zeros((RPS,), jnp.float32)

    h0, deg, agg1 = _make_agg_kernel(True)(x, n_id, edge_index, zrows, zdeg)
    h1 = _tc1_call(h0, agg1, deg, W1_self, W1_neigh, b1)
    agg2 = _make_agg_kernel(False)(h1, edge_index, zrows)
    return _tc2_call(h1, agg2, deg, W2_self, W2_neigh, b2,
                     W_lin1, b_lin1, W_lin2, b_lin2)
